# baseline (speedup 1.0000x reference)
import functools
import math

import jax
import jax.numpy as jnp
from jax.experimental import pallas as pl

N = 10000
E = 320000
BE = 3200
GE = E // BE


def _edge_dense_body(feat_ref, bank_ref, tx_ref, w14_ref, bee_ref,
                     wm1e_ref, bm1_ref, wm2e_ref, bm2_ref, wecc_ref, bec1_ref,
                     eaw1_ref, eaw2_ref, eac_ref):
    ea = feat_ref[...] @ w14_ref[...] + bank_ref[...] + tx_ref[...] + bee_ref[...]
    ea = jnp.maximum(ea, 0.0)
    eaw1_ref[...] = ea @ wm1e_ref[...] + bm1_ref[...]
    eaw2_ref[...] = ea @ wm2e_ref[...] + bm2_ref[...]
    eac_ref[...] = ea @ wecc_ref[...] + bec1_ref[...]


def _msg_body(xg_ref, eaw_ref, out_ref):
    out_ref[...] = jnp.maximum(xg_ref[...] + eaw_ref[...], 0.0)


def _node1_body(x_ref, agg_ref, deg_ref, wux_ref, wua_ref, bu_ref, wm2h_ref,
                h1_ref, h1w2_ref):
    agg = agg_ref[...] / (deg_ref[...] + 1e-6)
    h1 = jnp.maximum(x_ref[...] @ wux_ref[...] + agg @ wua_ref[...] + bu_ref[...], 0.0)
    h1_ref[...] = h1
    h1w2_ref[...] = h1 @ wm2h_ref[...]


def _node2_body(h1_ref, agg_ref, deg_ref, wux_ref, wua_ref, bu_ref,
                weca_ref, wecb_ref, emb_ref, emba_ref, embb_ref):
    agg = agg_ref[...] / (deg_ref[...] + 1e-6)
    emb = jnp.maximum(h1_ref[...] @ wux_ref[...] + agg @ wua_ref[...] + bu_ref[...], 0.0)
    emb_ref[...] = emb
    emba_ref[...] = emb @ weca_ref[...]
    embb_ref[...] = emb @ wecb_ref[...]


def _edge_head_body(ga_ref, gb_ref, eac_ref, wec2_ref, bec2_ref,
                    logit_ref, prob_ref):
    eh = jnp.maximum(ga_ref[...] + gb_ref[...] + eac_ref[...], 0.0)
    logit = eh @ wec2_ref[...] + bec2_ref[...]
    logit_ref[...] = logit
    prob_ref[...] = 1.0 / (1.0 + jnp.exp(-logit))


def _node_head_body(emb_ref, st_ref, wn1e_ref, wn1a_ref, bn1_ref, gam_ref,
                    bet_ref, wn2_ref, bn2_ref, out_ref):
    st = st_ref[...]
    cnt = st[:, 0:1]
    mean_prob = st[:, 1:2] / (cnt + 1e-6)
    maxp = st[:, 2:3]
    max_prob = jnp.where(maxp < -1e8, 0.0, maxp)
    count_high = jnp.log1p(st[:, 3:4])
    decay_weighted = st[:, 4:5] / (st[:, 5:6] + 1e-6)
    s30 = jnp.log1p(st[:, 6:7])
    m30v = st[:, 7:8]
    m30 = jnp.where(m30v < -1e8, 0.0, m30v)
    avg30 = st[:, 8:9] / (st[:, 9:10] + 1e-6)
    tsl = jnp.log1p(jnp.minimum(st[:, 10:11], 90.0)) * (1.0 / math.log1p(90.0))
    stats = (mean_prob, max_prob, count_high, decay_weighted, s30, m30,
             avg30, tsl)
    nh = emb_ref[...] @ wn1e_ref[...] + bn1_ref[...]
    for k, s in enumerate(stats):
        nh = nh + s * wn1a_ref[k:k + 1, :]
    nh = nh * (1.0 / math.sqrt(1.0 + 1e-5)) * gam_ref[...] + bet_ref[...]
    nh = jnp.maximum(nh, 0.0)
    out_ref[...] = nh @ wn2_ref[...] + bn2_ref[...]


BN = 2000
GN = N // BN


def _eblock(d):
    return pl.BlockSpec((BE, d), lambda i: (i, 0))


def _nblock(d):
    return pl.BlockSpec((BN, d), lambda i: (i, 0))


def _full2(a, b):
    return pl.BlockSpec((a, b), lambda i: (0, 0))


def _full1(a):
    return pl.BlockSpec((a,), lambda i: (0,))


def kernel(x, edge_index, edge_log_amount, edge_ts_encodings, edge_bank_pairs,
           edge_tx_types, edge_country_risks, edge_time_since_prevs,
           edge_time_gap_between_edges, edge_rolling_tx_count_7d,
           edge_rolling_tx_count_30d, edge_unix_ts, bank_emb, tx_emb, W_ee,
           b_ee, W_msg1, b_msg1, W_upd1, b_upd1, W_msg2, b_msg2, W_upd2,
           b_upd2, W_ec1, b_ec1, W_ec2, b_ec2, W_nc1, b_nc1, bn_gamma,
           bn_beta, W_nc2, b_nc2):
    f32 = jnp.float32
    src = edge_index[0]
    dst = edge_index[1]

    feat = jnp.concatenate([
        edge_log_amount[:, None], edge_country_risks[:, None],
        edge_time_since_prevs[:, None], edge_time_gap_between_edges[:, None],
        edge_rolling_tx_count_7d[:, None], edge_rolling_tx_count_30d[:, None],
        edge_ts_encodings], axis=1)
    bank_proj = bank_emb @ W_ee[14:22]
    tx_proj = tx_emb @ W_ee[22:26]
    bank_rows = bank_proj[edge_bank_pairs]
    tx_rows = tx_proj[edge_tx_types]

    eaw1, eaw2, eac = pl.pallas_call(
        _edge_dense_body,
        grid=(GE,),
        in_specs=[_eblock(14), _eblock(32), _eblock(32),
                  _full2(14, 32), _full1(32),
                  _full2(32, 128), _full1(128),
                  _full2(32, 64), _full1(64),
                  _full2(32, 64), _full1(64)],
        out_specs=[_eblock(128), _eblock(64), _eblock(64)],
        out_shape=[jax.ShapeDtypeStruct((E, 128), f32),
                   jax.ShapeDtypeStruct((E, 64), f32),
                   jax.ShapeDtypeStruct((E, 64), f32)],
    )(feat, bank_rows, tx_rows, W_ee[:14], b_ee,
      W_msg1[128:], b_msg1, W_msg2[128:], b_msg2, W_ec1[128:], b_ec1)

    xW1 = x @ W_msg1[:128]
    xg1 = xW1[src]
    msg1 = pl.pallas_call(
        _msg_body, grid=(GE,),
        in_specs=[_eblock(128), _eblock(128)],
        out_specs=_eblock(128),
        out_shape=jax.ShapeDtypeStruct((E, 128), f32),
    )(xg1, eaw1)
    agg1 = jnp.zeros((N, 128), f32).at[dst].add(msg1)
    deg = jnp.zeros((N,), f32).at[dst].add(1.0)
    degc = deg[:, None]

    h1, h1W2 = pl.pallas_call(
        _node1_body,
        grid=(GN,),
        in_specs=[_nblock(128), _nblock(128), _nblock(1),
                  _full2(128, 128), _full2(128, 128), _full1(128),
                  _full2(128, 64)],
        out_specs=[_nblock(128), _nblock(64)],
        out_shape=[jax.ShapeDtypeStruct((N, 128), f32),
                   jax.ShapeDtypeStruct((N, 64), f32)],
    )(x, agg1, degc, W_upd1[:128], W_upd1[128:], b_upd1, W_msg2[:128])

    hg2 = h1W2[src]
    msg2 = pl.pallas_call(
        _msg_body, grid=(GE,),
        in_specs=[_eblock(64), _eblock(64)],
        out_specs=_eblock(64),
        out_shape=jax.ShapeDtypeStruct((E, 64), f32),
    )(hg2, eaw2)
    agg2 = jnp.zeros((N, 64), f32).at[dst].add(msg2)

    emb, embA, embB = pl.pallas_call(
        _node2_body,
        grid=(GN,),
        in_specs=[_nblock(128), _nblock(64), _nblock(1),
                  _full2(128, 64), _full2(64, 64), _full1(64),
                  _full2(64, 64), _full2(64, 64)],
        out_specs=[_nblock(64), _nblock(64), _nblock(64)],
        out_shape=[jax.ShapeDtypeStruct((N, 64), f32)] * 3,
    )(h1, agg2, degc, W_upd2[:128], W_upd2[128:], b_upd2,
      W_ec1[:64], W_ec1[64:128])

    ga = embA[src]
    gb = embB[dst]
    edge_logits2, probs2 = pl.pallas_call(
        _edge_head_body, grid=(GE,),
        in_specs=[_eblock(64), _eblock(64), _eblock(64),
                  _full2(64, 1), _full2(1, 1)],
        out_specs=[_eblock(1)] * 2,
        out_shape=[jax.ShapeDtypeStruct((E, 1), f32)] * 2,
    )(ga, gb, eac, W_ec2, b_ec2[:, None])
    edge_logits = edge_logits2[:, 0]
    probs = probs2[:, 0]

    ts = edge_unix_ts.astype(f32)
    now = ts.max()
    age = jnp.maximum(now - ts, 0.0)
    decay = jnp.exp(-age / (30.0 * 86400.0))
    high = (probs >= 0.7).astype(f32)
    last30 = (age <= 30.0 * 86400.0).astype(f32)
    age_days = age / 86400.0
    minval = jnp.where(probs >= 0.7, age_days, jnp.inf)
    mvals = jnp.where(last30 > 0.5, probs, -1e9)

    def sc_add(v):
        return jnp.zeros((N,), f32).at[src].add(v).at[dst].add(v)

    def sc_max(v):
        return jnp.full((N,), -1e9, f32).at[src].max(v).at[dst].max(v)

    cnt = sc_add(jnp.ones_like(probs))
    sum_prob = sc_add(probs)
    max_prob = sc_max(probs)
    ch_raw = sc_add(high)
    ws = sc_add(probs * decay)
    wsum = sc_add(decay)
    s30_raw = sc_add(high * last30)
    m30 = sc_max(mvals)
    sr30 = sc_add(probs * last30)
    c30 = sc_add(last30)
    min_age = jnp.full((N,), 9999.0, f32).at[src].min(minval).at[dst].min(minval)

    stats11 = jnp.stack([cnt, sum_prob, max_prob, ch_raw, ws, wsum, s30_raw,
                         m30, sr30, c30, min_age], axis=1)
    node_logits2 = pl.pallas_call(
        _node_head_body,
        grid=(GN,),
        in_specs=[_nblock(64), _nblock(11),
                  _full2(64, 64), _full2(8, 64), _full1(64), _full1(64),
                  _full1(64), _full2(64, 1), _full2(1, 1)],
        out_specs=_nblock(1),
        out_shape=jax.ShapeDtypeStruct((N, 1), f32),
    )(emb, stats11,
      W_nc1[:64], W_nc1[64:], b_nc1, bn_gamma, bn_beta, W_nc2, b_nc2[:, None])

    return (node_logits2[:, 0], edge_logits)


# speedup vs baseline: 1.2287x; 1.2287x over previous
import functools
import math

import jax
import jax.numpy as jnp
from jax import lax
from jax.experimental import pallas as pl
from jax.experimental.pallas import tpu as pltpu
from jax.experimental.pallas import tpu_sc as plsc

N = 10000
E = 320000
BE = 3200
GE = E // BE


def _edge_dense_body(feat_ref, bank_ref, tx_ref, w14_ref, bee_ref,
                     wm1e_ref, bm1_ref, wm2e_ref, bm2_ref, wecc_ref, bec1_ref,
                     eaw1_ref, eaw2_ref, eac_ref):
    ea = feat_ref[...] @ w14_ref[...] + bank_ref[...] + tx_ref[...] + bee_ref[...]
    ea = jnp.maximum(ea, 0.0)
    eaw1_ref[...] = ea @ wm1e_ref[...] + bm1_ref[...]
    eaw2_ref[...] = ea @ wm2e_ref[...] + bm2_ref[...]
    eac_ref[...] = ea @ wecc_ref[...] + bec1_ref[...]


def _mm128_body(a_ref, w_ref, o_ref):
    o_ref[...] = a_ref[...] @ w_ref[...]


def _node1_body(x_ref, agg_ref, deg_ref, wux_ref, wua_ref, bu_ref, wm2h_ref,
                h1_ref, h1w2_ref):
    agg = agg_ref[...] / (deg_ref[...] + 1e-6)
    h1 = jnp.maximum(x_ref[...] @ wux_ref[...] + agg @ wua_ref[...] + bu_ref[...], 0.0)
    h1_ref[...] = h1
    h1w2_ref[...] = h1 @ wm2h_ref[...]


def _node2_body(h1_ref, agg_ref, deg_ref, wux_ref, wua_ref, bu_ref,
                weca_ref, wecb_ref, emb_ref, emba_ref, embb_ref):
    agg = agg_ref[...] / (deg_ref[...] + 1e-6)
    emb = jnp.maximum(h1_ref[...] @ wux_ref[...] + agg @ wua_ref[...] + bu_ref[...], 0.0)
    emb_ref[...] = emb
    emba_ref[...] = emb @ weca_ref[...]
    embb_ref[...] = emb @ wecb_ref[...]


def _edge_head_body(ga_ref, gb_ref, eac_ref, wec2_ref, bec2_ref,
                    logit_ref, prob_ref):
    eh = jnp.maximum(ga_ref[...] + gb_ref[...] + eac_ref[...], 0.0)
    logit = eh @ wec2_ref[...] + bec2_ref[...]
    logit_ref[...] = logit
    prob_ref[...] = 1.0 / (1.0 + jnp.exp(-logit))


def _node_head_body(emb_ref, st_ref, wn1e_ref, wn1a_ref, bn1_ref, gam_ref,
                    bet_ref, wn2_ref, bn2_ref, out_ref):
    st = st_ref[...]
    cnt = st[:, 0:1]
    mean_prob = st[:, 1:2] / (cnt + 1e-6)
    maxp = st[:, 2:3]
    max_prob = jnp.where(maxp < -1e8, 0.0, maxp)
    count_high = jnp.log1p(st[:, 3:4])
    decay_weighted = st[:, 4:5] / (st[:, 5:6] + 1e-6)
    s30 = jnp.log1p(st[:, 6:7])
    m30v = st[:, 7:8]
    m30 = jnp.where(m30v < -1e8, 0.0, m30v)
    avg30 = st[:, 8:9] / (st[:, 9:10] + 1e-6)
    tsl = jnp.log1p(jnp.minimum(st[:, 10:11], 90.0)) * (1.0 / math.log1p(90.0))
    stats = (mean_prob, max_prob, count_high, decay_weighted, s30, m30,
             avg30, tsl)
    nh = emb_ref[...] @ wn1e_ref[...] + bn1_ref[...]
    for k, s in enumerate(stats):
        nh = nh + s * wn1a_ref[k:k + 1, :]
    nh = nh * (1.0 / math.sqrt(1.0 + 1e-5)) * gam_ref[...] + bet_ref[...]
    nh = jnp.maximum(nh, 0.0)
    out_ref[...] = nh @ wn2_ref[...] + bn2_ref[...]


BN = 2000
GN = N // BN

_NC, _NS, _NW = 2, 16, 32
_EPT = E // _NW
_BCH = 128
_NFULL = _EPT // _BCH
_TAIL = _EPT - _NFULL * _BCH
_NPAD = 10240
_RPT = _NPAD // _NS
_RST = 128


def _make_sc_msg_agg(D, with_deg):
    f32 = jnp.float32
    mesh = plsc.VectorSubcoreMesh(core_axis_name="c", subcore_axis_name="s",
                                  num_cores=_NC, num_subcores=_NS)
    if with_deg:
        out_type = [jax.ShapeDtypeStruct((_NC, _NPAD, D), f32),
                    jax.ShapeDtypeStruct((_NC * N,), f32)]
    else:
        out_type = jax.ShapeDtypeStruct((_NC, _NPAD, D), f32)
    scratch = [
        pltpu.VMEM((_BCH,), jnp.int32),
        pltpu.VMEM((_BCH,), jnp.int32),
        pltpu.VMEM((_TAIL,), jnp.int32),
        pltpu.VMEM((_TAIL,), jnp.int32),
        pltpu.VMEM((_BCH, D), f32),
        pltpu.VMEM((_BCH, D), f32),
        pltpu.VMEM((_BCH,), f32),
        pltpu.VMEM((2000,), f32),
        pltpu.VMEM_SHARED((_NPAD, D), f32),
        pltpu.VMEM_SHARED((N,), f32),
        pltpu.SemaphoreType.DMA,
    ]

    def body(xw, eaw, srci, dsti, zrows, zdeg, ones, agg_out, *rest):
        if with_deg:
            deg_out = rest[0]
            rest = rest[1:]
        (src_v, dst_v, tsrc_v, tdst_v, g_v, ea_v, ones_v, dstage_v, agg_s,
         deg_s, sem) = rest
        cid = lax.axis_index("c")
        sid = lax.axis_index("s")
        wid = cid * _NS + sid
        base = wid * _EPT

        pltpu.sync_copy(zrows, g_v.at[pl.ds(0, _RST), :])
        for j in range(_RPT // _RST):
            pltpu.sync_copy(g_v.at[pl.ds(0, _RST), :],
                            agg_s.at[pl.ds(sid * _RPT + j * _RST, _RST), :])
        if with_deg:
            pltpu.sync_copy(ones, ones_v)

            @pl.when(sid == 0)
            def _():
                pltpu.sync_copy(zdeg, dstage_v)
                for j in range(N // 2000):
                    pltpu.sync_copy(dstage_v, deg_s.at[pl.ds(j * 2000, 2000)])
        plsc.subcore_barrier()

        def do_chunk(off, sz, sv, dv):
            pltpu.sync_copy(srci.at[pl.ds(base + off, sz)], sv)
            pltpu.sync_copy(dsti.at[pl.ds(base + off, sz)], dv)
            gd = g_v.at[pl.ds(0, sz), :] if sz != _BCH else g_v
            ed = ea_v.at[pl.ds(0, sz), :] if sz != _BCH else ea_v
            pltpu.async_copy(xw.at[sv], gd, sem).wait()
            pltpu.sync_copy(eaw.at[pl.ds(base + off, sz), :], ed)

            @pl.loop(0, sz)
            def _(r):
                for c in range(D // 16):
                    s = pl.ds(c * 16, 16)
                    g_v[r, s] = jnp.maximum(g_v[r, s] + ea_v[r, s], 0.0)

            pltpu.sync_copy(gd, agg_s.at[dv], add=True)
            if with_deg:
                ov = ones_v if sz == _BCH else ones_v.at[pl.ds(0, sz)]
                pltpu.sync_copy(ov, deg_s.at[dv], add=True)

        for ch in range(_NFULL):
            do_chunk(ch * _BCH, _BCH, src_v, dst_v)
        do_chunk(_NFULL * _BCH, _TAIL, tsrc_v, tdst_v)

        plsc.subcore_barrier()

        for j in range(_RPT // _RST):
            r0 = sid * _RPT + j * _RST
            pltpu.sync_copy(agg_s.at[pl.ds(r0, _RST), :],
                            g_v.at[pl.ds(0, _RST), :])
            pltpu.sync_copy(g_v.at[pl.ds(0, _RST), :],
                            agg_out.at[cid, pl.ds(r0, _RST), :])
        if with_deg:
            @pl.when(sid == 0)
            def _():
                for j in range(N // 2000):
                    pltpu.sync_copy(deg_s.at[pl.ds(j * 2000, 2000)], dstage_v)
                    pltpu.sync_copy(dstage_v,
                                    deg_out.at[pl.ds(cid * N + j * 2000, 2000)])

    return functools.partial(
        pl.kernel, out_type=out_type, mesh=mesh, scratch_types=scratch,
        compiler_params=pltpu.CompilerParams(use_tc_tiling_on_sc=False),
    )(body)


_sc_msg_agg_128 = _make_sc_msg_agg(128, True)
_sc_msg_agg_64 = _make_sc_msg_agg(64, False)


def _eblock(d):
    return pl.BlockSpec((BE, d), lambda i: (i, 0))


def _nblock(d):
    return pl.BlockSpec((BN, d), lambda i: (i, 0))


def _full2(a, b):
    return pl.BlockSpec((a, b), lambda i: (0, 0))


def _full1(a):
    return pl.BlockSpec((a,), lambda i: (0,))


def kernel(x, edge_index, edge_log_amount, edge_ts_encodings, edge_bank_pairs,
           edge_tx_types, edge_country_risks, edge_time_since_prevs,
           edge_time_gap_between_edges, edge_rolling_tx_count_7d,
           edge_rolling_tx_count_30d, edge_unix_ts, bank_emb, tx_emb, W_ee,
           b_ee, W_msg1, b_msg1, W_upd1, b_upd1, W_msg2, b_msg2, W_upd2,
           b_upd2, W_ec1, b_ec1, W_ec2, b_ec2, W_nc1, b_nc1, bn_gamma,
           bn_beta, W_nc2, b_nc2):
    f32 = jnp.float32
    src = edge_index[0]
    dst = edge_index[1]

    feat = jnp.concatenate([
        edge_log_amount[:, None], edge_country_risks[:, None],
        edge_time_since_prevs[:, None], edge_time_gap_between_edges[:, None],
        edge_rolling_tx_count_7d[:, None], edge_rolling_tx_count_30d[:, None],
        edge_ts_encodings], axis=1)
    bank_proj = bank_emb @ W_ee[14:22]
    tx_proj = tx_emb @ W_ee[22:26]
    bank_rows = bank_proj[edge_bank_pairs]
    tx_rows = tx_proj[edge_tx_types]

    eaw1, eaw2, eac = pl.pallas_call(
        _edge_dense_body,
        grid=(GE,),
        in_specs=[_eblock(14), _eblock(32), _eblock(32),
                  _full2(14, 32), _full1(32),
                  _full2(32, 128), _full1(128),
                  _full2(32, 64), _full1(64),
                  _full2(32, 64), _full1(64)],
        out_specs=[_eblock(128), _eblock(64), _eblock(64)],
        out_shape=[jax.ShapeDtypeStruct((E, 128), f32),
                   jax.ShapeDtypeStruct((E, 64), f32),
                   jax.ShapeDtypeStruct((E, 64), f32)],
    )(feat, bank_rows, tx_rows, W_ee[:14], b_ee,
      W_msg1[128:], b_msg1, W_msg2[128:], b_msg2, W_ec1[128:], b_ec1)

    xW1 = pl.pallas_call(
        _mm128_body, grid=(GN,),
        in_specs=[_nblock(128), _full2(128, 128)],
        out_specs=_nblock(128),
        out_shape=jax.ShapeDtypeStruct((N, 128), f32),
    )(x, W_msg1[:128])
    z128 = jnp.zeros((_RST, 128), f32)
    z64 = jnp.zeros((_RST, 64), f32)
    zN = jnp.zeros((2000,), f32)
    ones128 = jnp.ones((_BCH,), f32)
    agg1_p, deg_p = _sc_msg_agg_128(xW1, eaw1, src, dst, z128, zN, ones128)
    agg1 = agg1_p[0, :N] + agg1_p[1, :N]
    deg = deg_p[:N] + deg_p[N:]
    degc = deg[:, None]

    h1, h1W2 = pl.pallas_call(
        _node1_body,
        grid=(GN,),
        in_specs=[_nblock(128), _nblock(128), _nblock(1),
                  _full2(128, 128), _full2(128, 128), _full1(128),
                  _full2(128, 64)],
        out_specs=[_nblock(128), _nblock(64)],
        out_shape=[jax.ShapeDtypeStruct((N, 128), f32),
                   jax.ShapeDtypeStruct((N, 64), f32)],
    )(x, agg1, degc, W_upd1[:128], W_upd1[128:], b_upd1, W_msg2[:128])

    agg2_p = _sc_msg_agg_64(h1W2, eaw2, src, dst, z64, zN, ones128)
    agg2 = agg2_p[0, :N] + agg2_p[1, :N]

    emb, embA, embB = pl.pallas_call(
        _node2_body,
        grid=(GN,),
        in_specs=[_nblock(128), _nblock(64), _nblock(1),
                  _full2(128, 64), _full2(64, 64), _full1(64),
                  _full2(64, 64), _full2(64, 64)],
        out_specs=[_nblock(64), _nblock(64), _nblock(64)],
        out_shape=[jax.ShapeDtypeStruct((N, 64), f32)] * 3,
    )(h1, agg2, degc, W_upd2[:128], W_upd2[128:], b_upd2,
      W_ec1[:64], W_ec1[64:128])

    ga = embA[src]
    gb = embB[dst]
    edge_logits2, probs2 = pl.pallas_call(
        _edge_head_body, grid=(GE,),
        in_specs=[_eblock(64), _eblock(64), _eblock(64),
                  _full2(64, 1), _full2(1, 1)],
        out_specs=[_eblock(1)] * 2,
        out_shape=[jax.ShapeDtypeStruct((E, 1), f32)] * 2,
    )(ga, gb, eac, W_ec2, b_ec2[:, None])
    edge_logits = edge_logits2[:, 0]
    probs = probs2[:, 0]

    ts = edge_unix_ts.astype(f32)
    now = ts.max()
    age = jnp.maximum(now - ts, 0.0)
    decay = jnp.exp(-age / (30.0 * 86400.0))
    high = (probs >= 0.7).astype(f32)
    last30 = (age <= 30.0 * 86400.0).astype(f32)
    age_days = age / 86400.0
    minval = jnp.where(probs >= 0.7, age_days, jnp.inf)
    mvals = jnp.where(last30 > 0.5, probs, -1e9)

    def sc_add(v):
        return jnp.zeros((N,), f32).at[src].add(v).at[dst].add(v)

    def sc_max(v):
        return jnp.full((N,), -1e9, f32).at[src].max(v).at[dst].max(v)

    cnt = sc_add(jnp.ones_like(probs))
    sum_prob = sc_add(probs)
    max_prob = sc_max(probs)
    ch_raw = sc_add(high)
    ws = sc_add(probs * decay)
    wsum = sc_add(decay)
    s30_raw = sc_add(high * last30)
    m30 = sc_max(mvals)
    sr30 = sc_add(probs * last30)
    c30 = sc_add(last30)
    min_age = jnp.full((N,), 9999.0, f32).at[src].min(minval).at[dst].min(minval)

    stats11 = jnp.stack([cnt, sum_prob, max_prob, ch_raw, ws, wsum, s30_raw,
                         m30, sr30, c30, min_age], axis=1)
    node_logits2 = pl.pallas_call(
        _node_head_body,
        grid=(GN,),
        in_specs=[_nblock(64), _nblock(11),
                  _full2(64, 64), _full2(8, 64), _full1(64), _full1(64),
                  _full1(64), _full2(64, 1), _full2(1, 1)],
        out_specs=_nblock(1),
        out_shape=jax.ShapeDtypeStruct((N, 1), f32),
    )(emb, stats11,
      W_nc1[:64], W_nc1[64:], b_nc1, bn_gamma, bn_beta, W_nc2, b_nc2[:, None])

    return (node_logits2[:, 0], edge_logits)


# speedup vs baseline: 1.3311x; 1.0833x over previous
import functools
import math

import jax
import jax.numpy as jnp
from jax import lax
from jax.experimental import pallas as pl
from jax.experimental.pallas import tpu as pltpu
from jax.experimental.pallas import tpu_sc as plsc

N = 10000
E = 320000
BE = 3200
GE = E // BE


def _edge_dense_body(feat_ref, bank_ref, tx_ref, w14_ref, bee_ref,
                     wm1e_ref, bm1_ref, wm2e_ref, bm2_ref, wecc_ref, bec1_ref,
                     eaw1_ref, eaw2_ref, eac_ref):
    ea = feat_ref[...] @ w14_ref[...] + bank_ref[...] + tx_ref[...] + bee_ref[...]
    ea = jnp.maximum(ea, 0.0)
    eaw1_ref[...] = ea @ wm1e_ref[...] + bm1_ref[...]
    eaw2_ref[...] = ea @ wm2e_ref[...] + bm2_ref[...]
    eac_ref[...] = ea @ wecc_ref[...] + bec1_ref[...]


def _mm128_body(a_ref, w_ref, o_ref):
    o_ref[...] = a_ref[...] @ w_ref[...]


def _node1_body(x_ref, agg_ref, deg_ref, wux_ref, wua_ref, bu_ref, wm2h_ref,
                h1_ref, h1w2_ref):
    agg = agg_ref[...] / (deg_ref[...] + 1e-6)
    h1 = jnp.maximum(x_ref[...] @ wux_ref[...] + agg @ wua_ref[...] + bu_ref[...], 0.0)
    h1_ref[...] = h1
    h1w2_ref[...] = h1 @ wm2h_ref[...]


def _node2_body(h1_ref, agg_ref, deg_ref, wux_ref, wua_ref, bu_ref,
                weca_ref, wecb_ref, emb_ref, emba_ref, embb_ref):
    agg = agg_ref[...] / (deg_ref[...] + 1e-6)
    emb = jnp.maximum(h1_ref[...] @ wux_ref[...] + agg @ wua_ref[...] + bu_ref[...], 0.0)
    emb_ref[...] = emb
    emba_ref[...] = emb @ weca_ref[...]
    embb_ref[...] = emb @ wecb_ref[...]


def _edge_head_body(eh_ref, wec2_ref, bec2_ref, logit_ref, prob_ref):
    logit = eh_ref[...] @ wec2_ref[...] + bec2_ref[...]
    logit_ref[...] = logit
    prob_ref[...] = 1.0 / (1.0 + jnp.exp(-logit))


def _node_head_body(emb_ref, st_ref, wn1e_ref, wn1a_ref, bn1_ref, gam_ref,
                    bet_ref, wn2_ref, bn2_ref, out_ref):
    st = st_ref[...]
    cnt = st[:, 0:1]
    mean_prob = st[:, 1:2] / (cnt + 1e-6)
    maxp = st[:, 2:3]
    max_prob = jnp.where(maxp < -1e8, 0.0, maxp)
    count_high = jnp.log1p(st[:, 3:4])
    decay_weighted = st[:, 4:5] / (st[:, 5:6] + 1e-6)
    s30 = jnp.log1p(st[:, 6:7])
    m30v = st[:, 7:8]
    m30 = jnp.where(m30v < -1e8, 0.0, m30v)
    avg30 = st[:, 8:9] / (st[:, 9:10] + 1e-6)
    tsl = jnp.log1p(jnp.minimum(st[:, 10:11], 90.0)) * (1.0 / math.log1p(90.0))
    stats = (mean_prob, max_prob, count_high, decay_weighted, s30, m30,
             avg30, tsl)
    nh = emb_ref[...] @ wn1e_ref[...] + bn1_ref[...]
    for k, s in enumerate(stats):
        nh = nh + s * wn1a_ref[k:k + 1, :]
    nh = nh * (1.0 / math.sqrt(1.0 + 1e-5)) * gam_ref[...] + bet_ref[...]
    nh = jnp.maximum(nh, 0.0)
    out_ref[...] = nh @ wn2_ref[...] + bn2_ref[...]


BN = 2000
GN = N // BN

_NC, _NS, _NW = 2, 16, 32
_EPT = E // _NW
_BCH = 128
_NFULL = _EPT // _BCH
_TAIL = _EPT - _NFULL * _BCH
_NPAD = 10240
_RPT = _NPAD // _NS
_RST = 128


def _make_sc_msg_agg(D, with_deg):
    f32 = jnp.float32
    mesh = plsc.VectorSubcoreMesh(core_axis_name="c", subcore_axis_name="s",
                                  num_cores=_NC, num_subcores=_NS)
    if with_deg:
        out_type = [jax.ShapeDtypeStruct((_NC, _NPAD, D), f32),
                    jax.ShapeDtypeStruct((_NC * N,), f32)]
    else:
        out_type = jax.ShapeDtypeStruct((_NC, _NPAD, D), f32)
    scratch = [
        pltpu.VMEM((_BCH,), jnp.int32),
        pltpu.VMEM((_BCH,), jnp.int32),
        pltpu.VMEM((_TAIL,), jnp.int32),
        pltpu.VMEM((_TAIL,), jnp.int32),
        pltpu.VMEM((_BCH, D), f32),
        pltpu.VMEM((_BCH, D), f32),
        pltpu.VMEM((_BCH,), f32),
        pltpu.VMEM((2000,), f32),
        pltpu.VMEM_SHARED((_NPAD, D), f32),
        pltpu.VMEM_SHARED((N,), f32),
        pltpu.SemaphoreType.DMA,
    ]

    def body(xw, eaw, srci, dsti, zrows, zdeg, ones, agg_out, *rest):
        if with_deg:
            deg_out = rest[0]
            rest = rest[1:]
        (src_v, dst_v, tsrc_v, tdst_v, g_v, ea_v, ones_v, dstage_v, agg_s,
         deg_s, sem) = rest
        cid = lax.axis_index("c")
        sid = lax.axis_index("s")
        wid = cid * _NS + sid
        base = wid * _EPT

        pltpu.sync_copy(zrows, g_v.at[pl.ds(0, _RST), :])
        for j in range(_RPT // _RST):
            pltpu.sync_copy(g_v.at[pl.ds(0, _RST), :],
                            agg_s.at[pl.ds(sid * _RPT + j * _RST, _RST), :])
        if with_deg:
            pltpu.sync_copy(ones, ones_v)

            @pl.when(sid == 0)
            def _():
                pltpu.sync_copy(zdeg, dstage_v)
                for j in range(N // 2000):
                    pltpu.sync_copy(dstage_v, deg_s.at[pl.ds(j * 2000, 2000)])
        plsc.subcore_barrier()

        def do_chunk(off, sz, sv, dv):
            pltpu.sync_copy(srci.at[pl.ds(base + off, sz)], sv)
            pltpu.sync_copy(dsti.at[pl.ds(base + off, sz)], dv)
            gd = g_v.at[pl.ds(0, sz), :] if sz != _BCH else g_v
            ed = ea_v.at[pl.ds(0, sz), :] if sz != _BCH else ea_v
            pltpu.async_copy(xw.at[sv], gd, sem).wait()
            pltpu.sync_copy(eaw.at[pl.ds(base + off, sz), :], ed)

            @pl.loop(0, sz)
            def _(r):
                for c in range(D // 16):
                    s = pl.ds(c * 16, 16)
                    g_v[r, s] = jnp.maximum(g_v[r, s] + ea_v[r, s], 0.0)

            pltpu.sync_copy(gd, agg_s.at[dv], add=True)
            if with_deg:
                ov = ones_v if sz == _BCH else ones_v.at[pl.ds(0, sz)]
                pltpu.sync_copy(ov, deg_s.at[dv], add=True)

        for ch in range(_NFULL):
            do_chunk(ch * _BCH, _BCH, src_v, dst_v)
        do_chunk(_NFULL * _BCH, _TAIL, tsrc_v, tdst_v)

        plsc.subcore_barrier()

        for j in range(_RPT // _RST):
            r0 = sid * _RPT + j * _RST
            pltpu.sync_copy(agg_s.at[pl.ds(r0, _RST), :],
                            g_v.at[pl.ds(0, _RST), :])
            pltpu.sync_copy(g_v.at[pl.ds(0, _RST), :],
                            agg_out.at[cid, pl.ds(r0, _RST), :])
        if with_deg:
            @pl.when(sid == 0)
            def _():
                for j in range(N // 2000):
                    pltpu.sync_copy(deg_s.at[pl.ds(j * 2000, 2000)], dstage_v)
                    pltpu.sync_copy(dstage_v,
                                    deg_out.at[pl.ds(cid * N + j * 2000, 2000)])

    return functools.partial(
        pl.kernel, out_type=out_type, mesh=mesh, scratch_types=scratch,
        compiler_params=pltpu.CompilerParams(use_tc_tiling_on_sc=False),
    )(body)


_sc_msg_agg_128 = _make_sc_msg_agg(128, True)
_sc_msg_agg_64 = _make_sc_msg_agg(64, False)


def _make_sc_edge_eh():
    f32 = jnp.float32
    D = 64
    mesh = plsc.VectorSubcoreMesh(core_axis_name="c", subcore_axis_name="s",
                                  num_cores=_NC, num_subcores=_NS)
    scratch = [
        pltpu.VMEM((_BCH,), jnp.int32),
        pltpu.VMEM((_BCH,), jnp.int32),
        pltpu.VMEM((_TAIL,), jnp.int32),
        pltpu.VMEM((_TAIL,), jnp.int32),
        pltpu.VMEM((_BCH, D), f32),
        pltpu.VMEM((_BCH, D), f32),
        pltpu.VMEM((_BCH, D), f32),
        pltpu.SemaphoreType.DMA,
        pltpu.SemaphoreType.DMA,
    ]

    def body(emba, embb, eac, srci, dsti, eh_out,
             src_v, dst_v, tsrc_v, tdst_v, ga_v, gb_v, ec_v, sema, semb):
        cid = lax.axis_index("c")
        sid = lax.axis_index("s")
        wid = cid * _NS + sid
        base = wid * _EPT

        def do_chunk(off, sz, sv, dv):
            pltpu.sync_copy(srci.at[pl.ds(base + off, sz)], sv)
            pltpu.sync_copy(dsti.at[pl.ds(base + off, sz)], dv)
            gad = ga_v.at[pl.ds(0, sz), :] if sz != _BCH else ga_v
            gbd = gb_v.at[pl.ds(0, sz), :] if sz != _BCH else gb_v
            ecd = ec_v.at[pl.ds(0, sz), :] if sz != _BCH else ec_v
            cpa = pltpu.async_copy(emba.at[sv], gad, sema)
            cpb = pltpu.async_copy(embb.at[dv], gbd, semb)
            pltpu.sync_copy(eac.at[pl.ds(base + off, sz), :], ecd)
            cpa.wait()
            cpb.wait()

            @pl.loop(0, sz)
            def _(r):
                for c in range(D // 16):
                    s = pl.ds(c * 16, 16)
                    ga_v[r, s] = jnp.maximum(
                        ga_v[r, s] + gb_v[r, s] + ec_v[r, s], 0.0)

            pltpu.sync_copy(gad, eh_out.at[pl.ds(base + off, sz), :])

        for ch in range(_NFULL):
            do_chunk(ch * _BCH, _BCH, src_v, dst_v)
        do_chunk(_NFULL * _BCH, _TAIL, tsrc_v, tdst_v)

    return functools.partial(
        pl.kernel, out_type=jax.ShapeDtypeStruct((E, 64), f32), mesh=mesh,
        scratch_types=scratch,
        compiler_params=pltpu.CompilerParams(use_tc_tiling_on_sc=False),
    )(body)


_sc_edge_eh = _make_sc_edge_eh()


def _eblock(d):
    return pl.BlockSpec((BE, d), lambda i: (i, 0))


def _nblock(d):
    return pl.BlockSpec((BN, d), lambda i: (i, 0))


def _full2(a, b):
    return pl.BlockSpec((a, b), lambda i: (0, 0))


def _full1(a):
    return pl.BlockSpec((a,), lambda i: (0,))


def kernel(x, edge_index, edge_log_amount, edge_ts_encodings, edge_bank_pairs,
           edge_tx_types, edge_country_risks, edge_time_since_prevs,
           edge_time_gap_between_edges, edge_rolling_tx_count_7d,
           edge_rolling_tx_count_30d, edge_unix_ts, bank_emb, tx_emb, W_ee,
           b_ee, W_msg1, b_msg1, W_upd1, b_upd1, W_msg2, b_msg2, W_upd2,
           b_upd2, W_ec1, b_ec1, W_ec2, b_ec2, W_nc1, b_nc1, bn_gamma,
           bn_beta, W_nc2, b_nc2):
    f32 = jnp.float32
    src = edge_index[0]
    dst = edge_index[1]

    feat = jnp.concatenate([
        edge_log_amount[:, None], edge_country_risks[:, None],
        edge_time_since_prevs[:, None], edge_time_gap_between_edges[:, None],
        edge_rolling_tx_count_7d[:, None], edge_rolling_tx_count_30d[:, None],
        edge_ts_encodings], axis=1)
    bank_proj = bank_emb @ W_ee[14:22]
    tx_proj = tx_emb @ W_ee[22:26]
    bank_rows = bank_proj[edge_bank_pairs]
    tx_rows = tx_proj[edge_tx_types]

    eaw1, eaw2, eac = pl.pallas_call(
        _edge_dense_body,
        grid=(GE,),
        in_specs=[_eblock(14), _eblock(32), _eblock(32),
                  _full2(14, 32), _full1(32),
                  _full2(32, 128), _full1(128),
                  _full2(32, 64), _full1(64),
                  _full2(32, 64), _full1(64)],
        out_specs=[_eblock(128), _eblock(64), _eblock(64)],
        out_shape=[jax.ShapeDtypeStruct((E, 128), f32),
                   jax.ShapeDtypeStruct((E, 64), f32),
                   jax.ShapeDtypeStruct((E, 64), f32)],
    )(feat, bank_rows, tx_rows, W_ee[:14], b_ee,
      W_msg1[128:], b_msg1, W_msg2[128:], b_msg2, W_ec1[128:], b_ec1)

    xW1 = pl.pallas_call(
        _mm128_body, grid=(GN,),
        in_specs=[_nblock(128), _full2(128, 128)],
        out_specs=_nblock(128),
        out_shape=jax.ShapeDtypeStruct((N, 128), f32),
    )(x, W_msg1[:128])
    z128 = jnp.zeros((_RST, 128), f32)
    z64 = jnp.zeros((_RST, 64), f32)
    zN = jnp.zeros((2000,), f32)
    ones128 = jnp.ones((_BCH,), f32)
    agg1_p, deg_p = _sc_msg_agg_128(xW1, eaw1, src, dst, z128, zN, ones128)
    agg1 = agg1_p[0, :N] + agg1_p[1, :N]
    deg = deg_p[:N] + deg_p[N:]
    degc = deg[:, None]

    h1, h1W2 = pl.pallas_call(
        _node1_body,
        grid=(GN,),
        in_specs=[_nblock(128), _nblock(128), _nblock(1),
                  _full2(128, 128), _full2(128, 128), _full1(128),
                  _full2(128, 64)],
        out_specs=[_nblock(128), _nblock(64)],
        out_shape=[jax.ShapeDtypeStruct((N, 128), f32),
                   jax.ShapeDtypeStruct((N, 64), f32)],
    )(x, agg1, degc, W_upd1[:128], W_upd1[128:], b_upd1, W_msg2[:128])

    agg2_p = _sc_msg_agg_64(h1W2, eaw2, src, dst, z64, zN, ones128)
    agg2 = agg2_p[0, :N] + agg2_p[1, :N]

    emb, embA, embB = pl.pallas_call(
        _node2_body,
        grid=(GN,),
        in_specs=[_nblock(128), _nblock(64), _nblock(1),
                  _full2(128, 64), _full2(64, 64), _full1(64),
                  _full2(64, 64), _full2(64, 64)],
        out_specs=[_nblock(64), _nblock(64), _nblock(64)],
        out_shape=[jax.ShapeDtypeStruct((N, 64), f32)] * 3,
    )(h1, agg2, degc, W_upd2[:128], W_upd2[128:], b_upd2,
      W_ec1[:64], W_ec1[64:128])

    eh = _sc_edge_eh(embA, embB, eac, src, dst)
    edge_logits2, probs2 = pl.pallas_call(
        _edge_head_body, grid=(GE,),
        in_specs=[_eblock(64), _full2(64, 1), _full2(1, 1)],
        out_specs=[_eblock(1)] * 2,
        out_shape=[jax.ShapeDtypeStruct((E, 1), f32)] * 2,
    )(eh, W_ec2, b_ec2[:, None])
    edge_logits = edge_logits2[:, 0]
    probs = probs2[:, 0]

    ts = edge_unix_ts.astype(f32)
    now = ts.max()
    age = jnp.maximum(now - ts, 0.0)
    decay = jnp.exp(-age / (30.0 * 86400.0))
    high = (probs >= 0.7).astype(f32)
    last30 = (age <= 30.0 * 86400.0).astype(f32)
    age_days = age / 86400.0
    minval = jnp.where(probs >= 0.7, age_days, jnp.inf)
    mvals = jnp.where(last30 > 0.5, probs, -1e9)

    def sc_add(v):
        return jnp.zeros((N,), f32).at[src].add(v).at[dst].add(v)

    def sc_max(v):
        return jnp.full((N,), -1e9, f32).at[src].max(v).at[dst].max(v)

    cnt = sc_add(jnp.ones_like(probs))
    sum_prob = sc_add(probs)
    max_prob = sc_max(probs)
    ch_raw = sc_add(high)
    ws = sc_add(probs * decay)
    wsum = sc_add(decay)
    s30_raw = sc_add(high * last30)
    m30 = sc_max(mvals)
    sr30 = sc_add(probs * last30)
    c30 = sc_add(last30)
    min_age = jnp.full((N,), 9999.0, f32).at[src].min(minval).at[dst].min(minval)

    stats11 = jnp.stack([cnt, sum_prob, max_prob, ch_raw, ws, wsum, s30_raw,
                         m30, sr30, c30, min_age], axis=1)
    node_logits2 = pl.pallas_call(
        _node_head_body,
        grid=(GN,),
        in_specs=[_nblock(64), _nblock(11),
                  _full2(64, 64), _full2(8, 64), _full1(64), _full1(64),
                  _full1(64), _full2(64, 1), _full2(1, 1)],
        out_specs=_nblock(1),
        out_shape=jax.ShapeDtypeStruct((N, 1), f32),
    )(emb, stats11,
      W_nc1[:64], W_nc1[64:], b_nc1, bn_gamma, bn_beta, W_nc2, b_nc2[:, None])

    return (node_logits2[:, 0], edge_logits)


# speedup vs baseline: 1.4801x; 1.1120x over previous
import functools
import math

import jax
import jax.numpy as jnp
from jax import lax
from jax.experimental import pallas as pl
from jax.experimental.pallas import tpu as pltpu
from jax.experimental.pallas import tpu_sc as plsc

N = 10000
E = 320000
BE = 3200
GE = E // BE


def _edge_dense_body(feat_ref, bank_ref, tt_ref, txp_ref, w14_ref, bee_ref,
                     wm1e_ref, bm1_ref, wm2e_ref, bm2_ref, wecc_ref, bec1_ref,
                     eaw1_ref, eaw2_ref, eac_ref):
    tt = tt_ref[...]
    iota = lax.broadcasted_iota(jnp.int32, (tt.shape[0], 16), 1)
    onehot = (tt == iota).astype(jnp.float32)
    ea = (feat_ref[...] @ w14_ref[...] + bank_ref[...]
          + onehot @ txp_ref[...] + bee_ref[...])
    ea = jnp.maximum(ea, 0.0)
    eaw1_ref[...] = ea @ wm1e_ref[...] + bm1_ref[...]
    eaw2_ref[...] = ea @ wm2e_ref[...] + bm2_ref[...]
    eac_ref[...] = ea @ wecc_ref[...] + bec1_ref[...]


def _mm128_body(a_ref, w_ref, o_ref):
    o_ref[...] = a_ref[...] @ w_ref[...]


def _node1_body(x_ref, agg_ref, deg_ref, wux_ref, wua_ref, bu_ref, wm2h_ref,
                h1_ref, h1w2_ref):
    agg = agg_ref[...] / (deg_ref[...] + 1e-6)
    h1 = jnp.maximum(x_ref[...] @ wux_ref[...] + agg @ wua_ref[...] + bu_ref[...], 0.0)
    h1_ref[...] = h1
    h1w2_ref[...] = h1 @ wm2h_ref[...]


def _node2_body(h1_ref, agg_ref, deg_ref, wux_ref, wua_ref, bu_ref,
                weca_ref, wecb_ref, emb_ref, emba_ref, embb_ref):
    agg = agg_ref[...] / (deg_ref[...] + 1e-6)
    emb = jnp.maximum(h1_ref[...] @ wux_ref[...] + agg @ wua_ref[...] + bu_ref[...], 0.0)
    emb_ref[...] = emb
    emba_ref[...] = emb @ weca_ref[...]
    embb_ref[...] = emb @ wecb_ref[...]


def _edge_head_body(eh_ref, wec2_ref, bec2_ref, logit_ref, prob_ref):
    logit = eh_ref[...] @ wec2_ref[...] + bec2_ref[...]
    logit_ref[...] = logit
    prob_ref[...] = 1.0 / (1.0 + jnp.exp(-logit))


def _node_head_body(emb_ref, st_ref, wn1e_ref, wn1a_ref, bn1_ref, gam_ref,
                    bet_ref, wn2_ref, bn2_ref, out_ref):
    st = st_ref[...]
    cnt = st[:, 0:1]
    mean_prob = st[:, 1:2] / (cnt + 1e-6)
    maxp = st[:, 2:3]
    max_prob = jnp.where(maxp < -1e8, 0.0, maxp)
    count_high = jnp.log1p(st[:, 3:4])
    decay_weighted = st[:, 4:5] / (st[:, 5:6] + 1e-6)
    s30 = jnp.log1p(st[:, 6:7])
    m30v = st[:, 7:8]
    m30 = jnp.where(m30v < -1e8, 0.0, m30v)
    avg30 = st[:, 8:9] / (st[:, 9:10] + 1e-6)
    tsl = jnp.log1p(jnp.minimum(st[:, 10:11], 90.0)) * (1.0 / math.log1p(90.0))
    stats = (mean_prob, max_prob, count_high, decay_weighted, s30, m30,
             avg30, tsl)
    nh = emb_ref[...] @ wn1e_ref[...] + bn1_ref[...]
    for k, s in enumerate(stats):
        nh = nh + s * wn1a_ref[k:k + 1, :]
    nh = nh * (1.0 / math.sqrt(1.0 + 1e-5)) * gam_ref[...] + bet_ref[...]
    nh = jnp.maximum(nh, 0.0)
    out_ref[...] = nh @ wn2_ref[...] + bn2_ref[...]


BN = 2000
GN = N // BN

_NC, _NS, _NW = 2, 16, 32
_EPT = E // _NW
_BCH = 128
_NFULL = _EPT // _BCH
_TAIL = _EPT - _NFULL * _BCH
_NPAD = 10240
_RPT = _NPAD // _NS
_RST = 128


def _make_sc_msg_agg(D, with_deg):
    f32 = jnp.float32
    mesh = plsc.VectorSubcoreMesh(core_axis_name="c", subcore_axis_name="s",
                                  num_cores=_NC, num_subcores=_NS)
    if with_deg:
        out_type = [jax.ShapeDtypeStruct((_NC, _NPAD, D), f32),
                    jax.ShapeDtypeStruct((_NC * N,), f32)]
    else:
        out_type = jax.ShapeDtypeStruct((_NC, _NPAD, D), f32)
    scratch = [
        pltpu.VMEM((_BCH,), jnp.int32),
        pltpu.VMEM((_BCH,), jnp.int32),
        pltpu.VMEM((_TAIL,), jnp.int32),
        pltpu.VMEM((_TAIL,), jnp.int32),
        pltpu.VMEM((_BCH, D), f32),
        pltpu.VMEM((_BCH, D), f32),
        pltpu.VMEM((_BCH,), f32),
        pltpu.VMEM((2000,), f32),
        pltpu.VMEM_SHARED((_NPAD, D), f32),
        pltpu.VMEM_SHARED((N,), f32),
        pltpu.SemaphoreType.DMA,
    ]

    def body(xw, eaw, srci, dsti, zrows, zdeg, ones, agg_out, *rest):
        if with_deg:
            deg_out = rest[0]
            rest = rest[1:]
        (src_v, dst_v, tsrc_v, tdst_v, g_v, ea_v, ones_v, dstage_v, agg_s,
         deg_s, sem) = rest
        cid = lax.axis_index("c")
        sid = lax.axis_index("s")
        wid = cid * _NS + sid
        base = wid * _EPT

        pltpu.sync_copy(zrows, g_v.at[pl.ds(0, _RST), :])
        for j in range(_RPT // _RST):
            pltpu.sync_copy(g_v.at[pl.ds(0, _RST), :],
                            agg_s.at[pl.ds(sid * _RPT + j * _RST, _RST), :])
        if with_deg:
            pltpu.sync_copy(ones, ones_v)

            @pl.when(sid == 0)
            def _():
                pltpu.sync_copy(zdeg, dstage_v)
                for j in range(N // 2000):
                    pltpu.sync_copy(dstage_v, deg_s.at[pl.ds(j * 2000, 2000)])
        plsc.subcore_barrier()

        def do_chunk(off, sz, sv, dv):
            pltpu.sync_copy(srci.at[pl.ds(base + off, sz)], sv)
            pltpu.sync_copy(dsti.at[pl.ds(base + off, sz)], dv)
            gd = g_v.at[pl.ds(0, sz), :] if sz != _BCH else g_v
            ed = ea_v.at[pl.ds(0, sz), :] if sz != _BCH else ea_v
            pltpu.async_copy(xw.at[sv], gd, sem).wait()
            pltpu.sync_copy(eaw.at[pl.ds(base + off, sz), :], ed)

            @pl.loop(0, sz)
            def _(r):
                for c in range(D // 16):
                    s = pl.ds(c * 16, 16)
                    g_v[r, s] = jnp.maximum(g_v[r, s] + ea_v[r, s], 0.0)

            pltpu.sync_copy(gd, agg_s.at[dv], add=True)
            if with_deg:
                ov = ones_v if sz == _BCH else ones_v.at[pl.ds(0, sz)]
                pltpu.sync_copy(ov, deg_s.at[dv], add=True)

        for ch in range(_NFULL):
            do_chunk(ch * _BCH, _BCH, src_v, dst_v)
        do_chunk(_NFULL * _BCH, _TAIL, tsrc_v, tdst_v)

        plsc.subcore_barrier()

        for j in range(_RPT // _RST):
            r0 = sid * _RPT + j * _RST
            pltpu.sync_copy(agg_s.at[pl.ds(r0, _RST), :],
                            g_v.at[pl.ds(0, _RST), :])
            pltpu.sync_copy(g_v.at[pl.ds(0, _RST), :],
                            agg_out.at[cid, pl.ds(r0, _RST), :])
        if with_deg:
            @pl.when(sid == 0)
            def _():
                for j in range(N // 2000):
                    pltpu.sync_copy(deg_s.at[pl.ds(j * 2000, 2000)], dstage_v)
                    pltpu.sync_copy(dstage_v,
                                    deg_out.at[pl.ds(cid * N + j * 2000, 2000)])

    return functools.partial(
        pl.kernel, out_type=out_type, mesh=mesh, scratch_types=scratch,
        compiler_params=pltpu.CompilerParams(use_tc_tiling_on_sc=False),
    )(body)


_sc_msg_agg_128 = _make_sc_msg_agg(128, True)
_sc_msg_agg_64 = _make_sc_msg_agg(64, False)


def _make_sc_edge_eh():
    f32 = jnp.float32
    D = 64
    mesh = plsc.VectorSubcoreMesh(core_axis_name="c", subcore_axis_name="s",
                                  num_cores=_NC, num_subcores=_NS)
    scratch = [
        pltpu.VMEM((_BCH,), jnp.int32),
        pltpu.VMEM((_BCH,), jnp.int32),
        pltpu.VMEM((_TAIL,), jnp.int32),
        pltpu.VMEM((_TAIL,), jnp.int32),
        pltpu.VMEM((_BCH, D), f32),
        pltpu.VMEM((_BCH, D), f32),
        pltpu.VMEM((_BCH, D), f32),
        pltpu.SemaphoreType.DMA,
        pltpu.SemaphoreType.DMA,
    ]

    def body(emba, embb, eac, srci, dsti, eh_out,
             src_v, dst_v, tsrc_v, tdst_v, ga_v, gb_v, ec_v, sema, semb):
        cid = lax.axis_index("c")
        sid = lax.axis_index("s")
        wid = cid * _NS + sid
        base = wid * _EPT

        def do_chunk(off, sz, sv, dv):
            pltpu.sync_copy(srci.at[pl.ds(base + off, sz)], sv)
            pltpu.sync_copy(dsti.at[pl.ds(base + off, sz)], dv)
            gad = ga_v.at[pl.ds(0, sz), :] if sz != _BCH else ga_v
            gbd = gb_v.at[pl.ds(0, sz), :] if sz != _BCH else gb_v
            ecd = ec_v.at[pl.ds(0, sz), :] if sz != _BCH else ec_v
            cpa = pltpu.async_copy(emba.at[sv], gad, sema)
            cpb = pltpu.async_copy(embb.at[dv], gbd, semb)
            pltpu.sync_copy(eac.at[pl.ds(base + off, sz), :], ecd)
            cpa.wait()
            cpb.wait()

            @pl.loop(0, sz)
            def _(r):
                for c in range(D // 16):
                    s = pl.ds(c * 16, 16)
                    ga_v[r, s] = jnp.maximum(
                        ga_v[r, s] + gb_v[r, s] + ec_v[r, s], 0.0)

            pltpu.sync_copy(gad, eh_out.at[pl.ds(base + off, sz), :])

        for ch in range(_NFULL):
            do_chunk(ch * _BCH, _BCH, src_v, dst_v)
        do_chunk(_NFULL * _BCH, _TAIL, tsrc_v, tdst_v)

    return functools.partial(
        pl.kernel, out_type=jax.ShapeDtypeStruct((E, 64), f32), mesh=mesh,
        scratch_types=scratch,
        compiler_params=pltpu.CompilerParams(use_tc_tiling_on_sc=False),
    )(body)


_sc_edge_eh = _make_sc_edge_eh()


def _make_sc_gather32():
    f32 = jnp.float32
    D = 32
    mesh = plsc.VectorSubcoreMesh(core_axis_name="c", subcore_axis_name="s",
                                  num_cores=_NC, num_subcores=_NS)
    scratch = [
        pltpu.VMEM((_BCH,), jnp.int32),
        pltpu.VMEM((_TAIL,), jnp.int32),
        pltpu.VMEM((_BCH, D), f32),
        pltpu.SemaphoreType.DMA,
    ]

    def body(table, idxi, rows_out, idx_v, tidx_v, g_v, sem):
        cid = lax.axis_index("c")
        sid = lax.axis_index("s")
        wid = cid * _NS + sid
        base = wid * _EPT

        def do_chunk(off, sz, iv):
            pltpu.sync_copy(idxi.at[pl.ds(base + off, sz)], iv)
            gd = g_v.at[pl.ds(0, sz), :] if sz != _BCH else g_v
            pltpu.async_copy(table.at[iv], gd, sem).wait()
            pltpu.sync_copy(gd, rows_out.at[pl.ds(base + off, sz), :])

        for ch in range(_NFULL):
            do_chunk(ch * _BCH, _BCH, idx_v)
        do_chunk(_NFULL * _BCH, _TAIL, tidx_v)

    return functools.partial(
        pl.kernel, out_type=jax.ShapeDtypeStruct((E, 32), f32), mesh=mesh,
        scratch_types=scratch,
        compiler_params=pltpu.CompilerParams(use_tc_tiling_on_sc=False),
    )(body)


_sc_gather32 = _make_sc_gather32()


def _eblock(d):
    return pl.BlockSpec((BE, d), lambda i: (i, 0))


def _nblock(d):
    return pl.BlockSpec((BN, d), lambda i: (i, 0))


def _full2(a, b):
    return pl.BlockSpec((a, b), lambda i: (0, 0))


def _full1(a):
    return pl.BlockSpec((a,), lambda i: (0,))


def kernel(x, edge_index, edge_log_amount, edge_ts_encodings, edge_bank_pairs,
           edge_tx_types, edge_country_risks, edge_time_since_prevs,
           edge_time_gap_between_edges, edge_rolling_tx_count_7d,
           edge_rolling_tx_count_30d, edge_unix_ts, bank_emb, tx_emb, W_ee,
           b_ee, W_msg1, b_msg1, W_upd1, b_upd1, W_msg2, b_msg2, W_upd2,
           b_upd2, W_ec1, b_ec1, W_ec2, b_ec2, W_nc1, b_nc1, bn_gamma,
           bn_beta, W_nc2, b_nc2):
    f32 = jnp.float32
    src = edge_index[0]
    dst = edge_index[1]

    feat = jnp.concatenate([
        edge_log_amount[:, None], edge_country_risks[:, None],
        edge_time_since_prevs[:, None], edge_time_gap_between_edges[:, None],
        edge_rolling_tx_count_7d[:, None], edge_rolling_tx_count_30d[:, None],
        edge_ts_encodings], axis=1)
    bank_proj = bank_emb @ W_ee[14:22]
    tx_proj = tx_emb @ W_ee[22:26]
    bank_rows = _sc_gather32(bank_proj, edge_bank_pairs)
    tt2 = edge_tx_types[:, None]

    eaw1, eaw2, eac = pl.pallas_call(
        _edge_dense_body,
        grid=(GE,),
        in_specs=[_eblock(14), _eblock(32), _eblock(1),
                  _full2(16, 32),
                  _full2(14, 32), _full1(32),
                  _full2(32, 128), _full1(128),
                  _full2(32, 64), _full1(64),
                  _full2(32, 64), _full1(64)],
        out_specs=[_eblock(128), _eblock(64), _eblock(64)],
        out_shape=[jax.ShapeDtypeStruct((E, 128), f32),
                   jax.ShapeDtypeStruct((E, 64), f32),
                   jax.ShapeDtypeStruct((E, 64), f32)],
    )(feat, bank_rows, tt2, tx_proj, W_ee[:14], b_ee,
      W_msg1[128:], b_msg1, W_msg2[128:], b_msg2, W_ec1[128:], b_ec1)

    xW1 = pl.pallas_call(
        _mm128_body, grid=(GN,),
        in_specs=[_nblock(128), _full2(128, 128)],
        out_specs=_nblock(128),
        out_shape=jax.ShapeDtypeStruct((N, 128), f32),
    )(x, W_msg1[:128])
    z128 = jnp.zeros((_RST, 128), f32)
    z64 = jnp.zeros((_RST, 64), f32)
    zN = jnp.zeros((2000,), f32)
    ones128 = jnp.ones((_BCH,), f32)
    agg1_p, deg_p = _sc_msg_agg_128(xW1, eaw1, src, dst, z128, zN, ones128)
    agg1 = agg1_p[0, :N] + agg1_p[1, :N]
    deg = deg_p[:N] + deg_p[N:]
    degc = deg[:, None]

    h1, h1W2 = pl.pallas_call(
        _node1_body,
        grid=(GN,),
        in_specs=[_nblock(128), _nblock(128), _nblock(1),
                  _full2(128, 128), _full2(128, 128), _full1(128),
                  _full2(128, 64)],
        out_specs=[_nblock(128), _nblock(64)],
        out_shape=[jax.ShapeDtypeStruct((N, 128), f32),
                   jax.ShapeDtypeStruct((N, 64), f32)],
    )(x, agg1, degc, W_upd1[:128], W_upd1[128:], b_upd1, W_msg2[:128])

    agg2_p = _sc_msg_agg_64(h1W2, eaw2, src, dst, z64, zN, ones128)
    agg2 = agg2_p[0, :N] + agg2_p[1, :N]

    emb, embA, embB = pl.pallas_call(
        _node2_body,
        grid=(GN,),
        in_specs=[_nblock(128), _nblock(64), _nblock(1),
                  _full2(128, 64), _full2(64, 64), _full1(64),
                  _full2(64, 64), _full2(64, 64)],
        out_specs=[_nblock(64), _nblock(64), _nblock(64)],
        out_shape=[jax.ShapeDtypeStruct((N, 64), f32)] * 3,
    )(h1, agg2, degc, W_upd2[:128], W_upd2[128:], b_upd2,
      W_ec1[:64], W_ec1[64:128])

    eh = _sc_edge_eh(embA, embB, eac, src, dst)
    edge_logits2, probs2 = pl.pallas_call(
        _edge_head_body, grid=(GE,),
        in_specs=[_eblock(64), _full2(64, 1), _full2(1, 1)],
        out_specs=[_eblock(1)] * 2,
        out_shape=[jax.ShapeDtypeStruct((E, 1), f32)] * 2,
    )(eh, W_ec2, b_ec2[:, None])
    edge_logits = edge_logits2[:, 0]
    probs = probs2[:, 0]

    ts = edge_unix_ts.astype(f32)
    now = ts.max()
    age = jnp.maximum(now - ts, 0.0)
    decay = jnp.exp(-age / (30.0 * 86400.0))
    high = (probs >= 0.7).astype(f32)
    last30 = (age <= 30.0 * 86400.0).astype(f32)
    age_days = age / 86400.0
    minval = jnp.where(probs >= 0.7, age_days, jnp.inf)
    mvals = jnp.where(last30 > 0.5, probs, -1e9)

    def sc_add(v):
        return jnp.zeros((N,), f32).at[src].add(v).at[dst].add(v)

    def sc_max(v):
        return jnp.full((N,), -1e9, f32).at[src].max(v).at[dst].max(v)

    cnt = sc_add(jnp.ones_like(probs))
    sum_prob = sc_add(probs)
    max_prob = sc_max(probs)
    ch_raw = sc_add(high)
    ws = sc_add(probs * decay)
    wsum = sc_add(decay)
    s30_raw = sc_add(high * last30)
    m30 = sc_max(mvals)
    sr30 = sc_add(probs * last30)
    c30 = sc_add(last30)
    min_age = jnp.full((N,), 9999.0, f32).at[src].min(minval).at[dst].min(minval)

    stats11 = jnp.stack([cnt, sum_prob, max_prob, ch_raw, ws, wsum, s30_raw,
                         m30, sr30, c30, min_age], axis=1)
    node_logits2 = pl.pallas_call(
        _node_head_body,
        grid=(GN,),
        in_specs=[_nblock(64), _nblock(11),
                  _full2(64, 64), _full2(8, 64), _full1(64), _full1(64),
                  _full1(64), _full2(64, 1), _full2(1, 1)],
        out_specs=_nblock(1),
        out_shape=jax.ShapeDtypeStruct((N, 1), f32),
    )(emb, stats11,
      W_nc1[:64], W_nc1[64:], b_nc1, bn_gamma, bn_beta, W_nc2, b_nc2[:, None])

    return (node_logits2[:, 0], edge_logits)


# speedup vs baseline: 3.8676x; 2.6131x over previous
import functools
import math

import jax
import jax.numpy as jnp
from jax import lax
from jax.experimental import pallas as pl
from jax.experimental.pallas import tpu as pltpu
from jax.experimental.pallas import tpu_sc as plsc

N = 10000
E = 320000
BE = 3200
GE = E // BE


def _edge_dense_body(feat_ref, bank_ref, tt_ref, txp_ref, w14_ref, bee_ref,
                     wm1e_ref, bm1_ref, wm2e_ref, bm2_ref, wecc_ref, bec1_ref,
                     eaw1_ref, eaw2_ref, eac_ref):
    tt = tt_ref[...]
    iota = lax.broadcasted_iota(jnp.int32, (tt.shape[0], 16), 1)
    onehot = (tt == iota).astype(jnp.float32)
    ea = (feat_ref[...] @ w14_ref[...] + bank_ref[...]
          + onehot @ txp_ref[...] + bee_ref[...])
    ea = jnp.maximum(ea, 0.0)
    eaw1_ref[...] = ea @ wm1e_ref[...] + bm1_ref[...]
    eaw2_ref[...] = ea @ wm2e_ref[...] + bm2_ref[...]
    eac_ref[...] = ea @ wecc_ref[...] + bec1_ref[...]


def _mm128_body(a_ref, w_ref, o_ref):
    o_ref[...] = a_ref[...] @ w_ref[...]


def _node1_body(x_ref, agg_ref, deg_ref, wux_ref, wua_ref, bu_ref, wm2h_ref,
                h1_ref, h1w2_ref):
    agg = agg_ref[...] / (deg_ref[...] + 1e-6)
    h1 = jnp.maximum(x_ref[...] @ wux_ref[...] + agg @ wua_ref[...] + bu_ref[...], 0.0)
    h1_ref[...] = h1
    h1w2_ref[...] = h1 @ wm2h_ref[...]


def _node2_body(h1_ref, agg_ref, deg_ref, wux_ref, wua_ref, bu_ref,
                weca_ref, wecb_ref, emb_ref, emba_ref, embb_ref):
    agg = agg_ref[...] / (deg_ref[...] + 1e-6)
    emb = jnp.maximum(h1_ref[...] @ wux_ref[...] + agg @ wua_ref[...] + bu_ref[...], 0.0)
    emb_ref[...] = emb
    emba_ref[...] = emb @ weca_ref[...]
    embb_ref[...] = emb @ wecb_ref[...]


def _edge_head_body(eh_ref, wec2_ref, bec2_ref, ts_ref, now_ref,
                    logit_ref, prob_ref, decay_ref, l30_ref, aged_ref):
    logit = eh_ref[...] @ wec2_ref[...] + bec2_ref[...]
    logit_ref[...] = logit
    prob_ref[...] = 1.0 / (1.0 + jnp.exp(-logit))
    age = jnp.maximum(now_ref[...] - ts_ref[...], 0.0)
    decay_ref[...] = jnp.exp(age * (-1.0 / (30.0 * 86400.0)))
    l30_ref[...] = jnp.where(age <= 30.0 * 86400.0, 1.0, 0.0)
    aged_ref[...] = age * (1.0 / 86400.0)


def _node_head_body(emb_ref, st_ref, wn1e_ref, wn1a_ref, bn1_ref, gam_ref,
                    bet_ref, wn2_ref, bn2_ref, out_ref):
    st = st_ref[...]
    cnt = st[:, 0:1]
    mean_prob = st[:, 1:2] / (cnt + 1e-6)
    maxp = st[:, 2:3]
    max_prob = jnp.where(maxp < -1e8, 0.0, maxp)
    count_high = jnp.log1p(st[:, 3:4])
    decay_weighted = st[:, 4:5] / (st[:, 5:6] + 1e-6)
    s30 = jnp.log1p(st[:, 6:7])
    m30v = st[:, 7:8]
    m30 = jnp.where(m30v < -1e8, 0.0, m30v)
    avg30 = st[:, 8:9] / (st[:, 9:10] + 1e-6)
    tsl = jnp.log1p(jnp.minimum(st[:, 10:11], 90.0)) * (1.0 / math.log1p(90.0))
    stats = (mean_prob, max_prob, count_high, decay_weighted, s30, m30,
             avg30, tsl)
    nh = emb_ref[...] @ wn1e_ref[...] + bn1_ref[...]
    for k, s in enumerate(stats):
        nh = nh + s * wn1a_ref[k:k + 1, :]
    nh = nh * (1.0 / math.sqrt(1.0 + 1e-5)) * gam_ref[...] + bet_ref[...]
    nh = jnp.maximum(nh, 0.0)
    out_ref[...] = nh @ wn2_ref[...] + bn2_ref[...]


BN = 2000
GN = N // BN

_NC, _NS, _NW = 2, 16, 32
_EPT = E // _NW
_BCH = 128
_NFULL = _EPT // _BCH
_TAIL = _EPT - _NFULL * _BCH
_NPAD = 10240
_RPT = _NPAD // _NS
_RST = 128


def _make_sc_msg_agg(D, with_deg):
    f32 = jnp.float32
    mesh = plsc.VectorSubcoreMesh(core_axis_name="c", subcore_axis_name="s",
                                  num_cores=_NC, num_subcores=_NS)
    if with_deg:
        out_type = [jax.ShapeDtypeStruct((_NC, _NPAD, D), f32),
                    jax.ShapeDtypeStruct((_NC * N,), f32)]
    else:
        out_type = jax.ShapeDtypeStruct((_NC, _NPAD, D), f32)
    scratch = [
        pltpu.VMEM((_BCH,), jnp.int32),
        pltpu.VMEM((_BCH,), jnp.int32),
        pltpu.VMEM((_TAIL,), jnp.int32),
        pltpu.VMEM((_TAIL,), jnp.int32),
        pltpu.VMEM((_BCH, D), f32),
        pltpu.VMEM((_BCH, D), f32),
        pltpu.VMEM((_BCH,), f32),
        pltpu.VMEM((2000,), f32),
        pltpu.VMEM_SHARED((_NPAD, D), f32),
        pltpu.VMEM_SHARED((N,), f32),
        pltpu.SemaphoreType.DMA,
    ]

    def body(xw, eaw, srci, dsti, zrows, zdeg, ones, agg_out, *rest):
        if with_deg:
            deg_out = rest[0]
            rest = rest[1:]
        (src_v, dst_v, tsrc_v, tdst_v, g_v, ea_v, ones_v, dstage_v, agg_s,
         deg_s, sem) = rest
        cid = lax.axis_index("c")
        sid = lax.axis_index("s")
        wid = cid * _NS + sid
        base = wid * _EPT

        pltpu.sync_copy(zrows, g_v.at[pl.ds(0, _RST), :])
        for j in range(_RPT // _RST):
            pltpu.sync_copy(g_v.at[pl.ds(0, _RST), :],
                            agg_s.at[pl.ds(sid * _RPT + j * _RST, _RST), :])
        if with_deg:
            pltpu.sync_copy(ones, ones_v)

            @pl.when(sid == 0)
            def _():
                pltpu.sync_copy(zdeg, dstage_v)
                for j in range(N // 2000):
                    pltpu.sync_copy(dstage_v, deg_s.at[pl.ds(j * 2000, 2000)])
        plsc.subcore_barrier()

        def do_chunk(off, sz, sv, dv):
            pltpu.sync_copy(srci.at[pl.ds(base + off, sz)], sv)
            pltpu.sync_copy(dsti.at[pl.ds(base + off, sz)], dv)
            gd = g_v.at[pl.ds(0, sz), :] if sz != _BCH else g_v
            ed = ea_v.at[pl.ds(0, sz), :] if sz != _BCH else ea_v
            pltpu.async_copy(xw.at[sv], gd, sem).wait()
            pltpu.sync_copy(eaw.at[pl.ds(base + off, sz), :], ed)

            @pl.loop(0, sz)
            def _(r):
                for c in range(D // 16):
                    s = pl.ds(c * 16, 16)
                    g_v[r, s] = jnp.maximum(g_v[r, s] + ea_v[r, s], 0.0)

            pltpu.sync_copy(gd, agg_s.at[dv], add=True)
            if with_deg:
                ov = ones_v if sz == _BCH else ones_v.at[pl.ds(0, sz)]
                pltpu.sync_copy(ov, deg_s.at[dv], add=True)

        for ch in range(_NFULL):
            do_chunk(ch * _BCH, _BCH, src_v, dst_v)
        do_chunk(_NFULL * _BCH, _TAIL, tsrc_v, tdst_v)

        plsc.subcore_barrier()

        for j in range(_RPT // _RST):
            r0 = sid * _RPT + j * _RST
            pltpu.sync_copy(agg_s.at[pl.ds(r0, _RST), :],
                            g_v.at[pl.ds(0, _RST), :])
            pltpu.sync_copy(g_v.at[pl.ds(0, _RST), :],
                            agg_out.at[cid, pl.ds(r0, _RST), :])
        if with_deg:
            @pl.when(sid == 0)
            def _():
                for j in range(N // 2000):
                    pltpu.sync_copy(deg_s.at[pl.ds(j * 2000, 2000)], dstage_v)
                    pltpu.sync_copy(dstage_v,
                                    deg_out.at[pl.ds(cid * N + j * 2000, 2000)])

    return functools.partial(
        pl.kernel, out_type=out_type, mesh=mesh, scratch_types=scratch,
        compiler_params=pltpu.CompilerParams(use_tc_tiling_on_sc=False),
    )(body)


_sc_msg_agg_128 = _make_sc_msg_agg(128, True)
_sc_msg_agg_64 = _make_sc_msg_agg(64, False)


def _make_sc_edge_eh():
    f32 = jnp.float32
    D = 64
    mesh = plsc.VectorSubcoreMesh(core_axis_name="c", subcore_axis_name="s",
                                  num_cores=_NC, num_subcores=_NS)
    scratch = [
        pltpu.VMEM((_BCH,), jnp.int32),
        pltpu.VMEM((_BCH,), jnp.int32),
        pltpu.VMEM((_TAIL,), jnp.int32),
        pltpu.VMEM((_TAIL,), jnp.int32),
        pltpu.VMEM((_BCH, D), f32),
        pltpu.VMEM((_BCH, D), f32),
        pltpu.VMEM((_BCH, D), f32),
        pltpu.SemaphoreType.DMA,
        pltpu.SemaphoreType.DMA,
    ]

    def body(emba, embb, eac, srci, dsti, eh_out,
             src_v, dst_v, tsrc_v, tdst_v, ga_v, gb_v, ec_v, sema, semb):
        cid = lax.axis_index("c")
        sid = lax.axis_index("s")
        wid = cid * _NS + sid
        base = wid * _EPT

        def do_chunk(off, sz, sv, dv):
            pltpu.sync_copy(srci.at[pl.ds(base + off, sz)], sv)
            pltpu.sync_copy(dsti.at[pl.ds(base + off, sz)], dv)
            gad = ga_v.at[pl.ds(0, sz), :] if sz != _BCH else ga_v
            gbd = gb_v.at[pl.ds(0, sz), :] if sz != _BCH else gb_v
            ecd = ec_v.at[pl.ds(0, sz), :] if sz != _BCH else ec_v
            cpa = pltpu.async_copy(emba.at[sv], gad, sema)
            cpb = pltpu.async_copy(embb.at[dv], gbd, semb)
            pltpu.sync_copy(eac.at[pl.ds(base + off, sz), :], ecd)
            cpa.wait()
            cpb.wait()

            @pl.loop(0, sz)
            def _(r):
                for c in range(D // 16):
                    s = pl.ds(c * 16, 16)
                    ga_v[r, s] = jnp.maximum(
                        ga_v[r, s] + gb_v[r, s] + ec_v[r, s], 0.0)

            pltpu.sync_copy(gad, eh_out.at[pl.ds(base + off, sz), :])

        for ch in range(_NFULL):
            do_chunk(ch * _BCH, _BCH, src_v, dst_v)
        do_chunk(_NFULL * _BCH, _TAIL, tsrc_v, tdst_v)

    return functools.partial(
        pl.kernel, out_type=jax.ShapeDtypeStruct((E, 64), f32), mesh=mesh,
        scratch_types=scratch,
        compiler_params=pltpu.CompilerParams(use_tc_tiling_on_sc=False),
    )(body)


_sc_edge_eh = _make_sc_edge_eh()


def _make_sc_gather32():
    f32 = jnp.float32
    D = 32
    mesh = plsc.VectorSubcoreMesh(core_axis_name="c", subcore_axis_name="s",
                                  num_cores=_NC, num_subcores=_NS)
    scratch = [
        pltpu.VMEM((_BCH,), jnp.int32),
        pltpu.VMEM((_TAIL,), jnp.int32),
        pltpu.VMEM((_BCH, D), f32),
        pltpu.SemaphoreType.DMA,
    ]

    def body(table, idxi, rows_out, idx_v, tidx_v, g_v, sem):
        cid = lax.axis_index("c")
        sid = lax.axis_index("s")
        wid = cid * _NS + sid
        base = wid * _EPT

        def do_chunk(off, sz, iv):
            pltpu.sync_copy(idxi.at[pl.ds(base + off, sz)], iv)
            gd = g_v.at[pl.ds(0, sz), :] if sz != _BCH else g_v
            pltpu.async_copy(table.at[iv], gd, sem).wait()
            pltpu.sync_copy(gd, rows_out.at[pl.ds(base + off, sz), :])

        for ch in range(_NFULL):
            do_chunk(ch * _BCH, _BCH, idx_v)
        do_chunk(_NFULL * _BCH, _TAIL, tidx_v)

    return functools.partial(
        pl.kernel, out_type=jax.ShapeDtypeStruct((E, 32), f32), mesh=mesh,
        scratch_types=scratch,
        compiler_params=pltpu.CompilerParams(use_tc_tiling_on_sc=False),
    )(body)


_sc_gather32 = _make_sc_gather32()


def _make_sc_stats():
    f32 = jnp.float32
    i32 = jnp.int32
    RPT = _NPAD // _NS

    mesh = plsc.VectorSubcoreMesh(core_axis_name="c", subcore_axis_name="s",
                                  num_cores=_NC, num_subcores=_NS)
    scratch = [
        pltpu.VMEM((_BCH,), i32),
        pltpu.VMEM((_BCH,), i32),
        pltpu.VMEM((_TAIL,), i32),
        pltpu.VMEM((_TAIL,), i32),
        pltpu.VMEM((_BCH,), f32),
        pltpu.VMEM((_BCH,), f32),
        pltpu.VMEM((_BCH,), f32),
        pltpu.VMEM((_BCH,), f32),
        pltpu.VMEM((_BCH, 8), f32),
        pltpu.VMEM((_NPAD,), f32),
        pltpu.VMEM((_NPAD,), f32),
        pltpu.VMEM((_NPAD,), f32),
        pltpu.VMEM((_NS, RPT), f32),
        pltpu.VMEM((RPT,), f32),
        pltpu.VMEM((RPT, 8), f32),
        pltpu.VMEM((48,), i32),
        pltpu.VMEM((48,), f32),
        pltpu.VMEM_SHARED((_NPAD, 8), f32),
        pltpu.VMEM_SHARED((3, _NS, _NPAD), f32),
        pltpu.SemaphoreType.DMA,
    ]

    def body(probs, decay, last30, aged, srci, dsti, z8,
             adds_out, mm_out,
             src_v, dst_v, tsrc_v, tdst_v, p_v, dc_v, l3_v, ag_v, rows_v,
             mx_v, m3_v, mn_v, red_v, out_v, zst_v, kw_v, vw_v,
             adds_s, mm_s, sem):
        cid = lax.axis_index("c")
        sid = lax.axis_index("s")
        wid = cid * _NS + sid
        base = wid * _EPT
        iota = lax.iota(i32, 16)

        pltpu.sync_copy(z8, zst_v)
        pltpu.sync_copy(zst_v, adds_s.at[pl.ds(sid * RPT, RPT), :])

        @pl.loop(0, _NPAD // 16)
        def _(i):
            s = pl.ds(i * 16, 16)
            mx_v[s] = jnp.full((16,), -1e9, f32)
            m3_v[s] = jnp.full((16,), -1e9, f32)
            mn_v[s] = jnp.full((16,), 9999.0, f32)

        kw_v[pl.ds(0, 16)] = jnp.full((16,), -1, i32)
        kw_v[pl.ds(32, 16)] = jnp.full((16,), -2, i32)
        plsc.subcore_barrier()

        def segreduce(arr, kv, vv, is_min):
            ks, vs = plsc.sort_key_val(kv, vv)
            op = jnp.minimum if is_min else jnp.maximum
            kw_v[pl.ds(16, 16)] = ks
            for sh in (1, 2, 4, 8):
                vw_v[pl.ds(16, 16)] = vs
                same = kw_v[pl.ds(16 - sh, 16)] == ks
                shifted = vw_v[pl.ds(16 - sh, 16)]
                vs = jnp.where(same, op(vs, shifted), vs)
            knext = kw_v[pl.ds(17, 16)]
            last = knext != ks
            cur = plsc.load_gather(arr, [ks], mask=last)
            plsc.store_scatter(arr, [ks], op(cur, vs), mask=last)

        def do_chunk(off, sz, sv, dv):
            pltpu.sync_copy(srci.at[pl.ds(base + off, sz)], sv)
            pltpu.sync_copy(dsti.at[pl.ds(base + off, sz)], dv)
            pltpu.sync_copy(probs.at[pl.ds(base + off, sz)],
                            p_v.at[pl.ds(0, sz)])
            pltpu.sync_copy(decay.at[pl.ds(base + off, sz)],
                            dc_v.at[pl.ds(0, sz)])
            pltpu.sync_copy(last30.at[pl.ds(base + off, sz)],
                            l3_v.at[pl.ds(0, sz)])
            pltpu.sync_copy(aged.at[pl.ds(base + off, sz)],
                            ag_v.at[pl.ds(0, sz)])
            @pl.loop(0, sz // 16)
            def _(g):
                sl = pl.ds(g * 16, 16)
                eids = iota + g * 16
                p16 = p_v[sl]
                dc16 = dc_v[sl]
                l316 = l3_v[sl]
                ag16 = ag_v[sl]
                high = jnp.where(p16 >= 0.7, 1.0, 0.0).astype(f32)
                vals = (jnp.full((16,), 1.0, f32), p16, high, p16 * dc16,
                        dc16, high * l316, p16 * l316, l316)
                for k, v in enumerate(vals):
                    plsc.store_scatter(rows_v, [eids, jnp.full((16,), k, i32)], v)
                vm30 = jnp.where(l316 > 0.5, p16, jnp.full((16,), -1e9, f32))
                vmin = jnp.where(p16 >= 0.7, ag16, jnp.full((16,), 1e30, f32))
                for kv in (sv[sl], dv[sl]):
                    segreduce(mx_v, kv, p16, False)
                    segreduce(m3_v, kv, vm30, False)
                    segreduce(mn_v, kv, vmin, True)
            rd = rows_v if sz == _BCH else rows_v.at[pl.ds(0, sz), :]
            pltpu.sync_copy(rd, adds_s.at[sv], add=True)
            pltpu.sync_copy(rd, adds_s.at[dv], add=True)

        @pl.loop(0, _NFULL)
        def _(ch):
            do_chunk(pl.multiple_of(ch * _BCH, _BCH), _BCH, src_v, dst_v)

        do_chunk(_NFULL * _BCH, _TAIL, tsrc_v, tdst_v)

        pltpu.sync_copy(mx_v, mm_s.at[0, sid, :])
        pltpu.sync_copy(m3_v, mm_s.at[1, sid, :])
        pltpu.sync_copy(mn_v, mm_s.at[2, sid, :])
        plsc.subcore_barrier()

        pltpu.sync_copy(adds_s.at[pl.ds(sid * RPT, RPT), :], zst_v)
        pltpu.sync_copy(zst_v, adds_out.at[cid, pl.ds(sid * RPT, RPT), :])
        for st in range(3):
            pltpu.sync_copy(mm_s.at[st, :, pl.ds(sid * RPT, RPT)], red_v)
            op = jnp.minimum if st == 2 else jnp.maximum

            @pl.loop(0, RPT // 16)
            def _(j):
                sl = pl.ds(j * 16, 16)
                acc = red_v[0, sl]
                for r in range(1, _NS):
                    acc = op(acc, red_v[r, sl])
                out_v[sl] = acc

            pltpu.sync_copy(out_v, mm_out.at[cid, st, pl.ds(sid * RPT, RPT)])

    return functools.partial(
        pl.kernel,
        out_type=[jax.ShapeDtypeStruct((_NC, _NPAD, 8), f32),
                  jax.ShapeDtypeStruct((_NC, 3, _NPAD), f32)],
        mesh=mesh, scratch_types=scratch,
        compiler_params=pltpu.CompilerParams(use_tc_tiling_on_sc=False,
                                             needs_layout_passes=False),
    )(body)


_sc_stats = _make_sc_stats()


def _eblock(d):
    return pl.BlockSpec((BE, d), lambda i: (i, 0))


def _nblock(d):
    return pl.BlockSpec((BN, d), lambda i: (i, 0))


def _full2(a, b):
    return pl.BlockSpec((a, b), lambda i: (0, 0))


def _full1(a):
    return pl.BlockSpec((a,), lambda i: (0,))


def kernel(x, edge_index, edge_log_amount, edge_ts_encodings, edge_bank_pairs,
           edge_tx_types, edge_country_risks, edge_time_since_prevs,
           edge_time_gap_between_edges, edge_rolling_tx_count_7d,
           edge_rolling_tx_count_30d, edge_unix_ts, bank_emb, tx_emb, W_ee,
           b_ee, W_msg1, b_msg1, W_upd1, b_upd1, W_msg2, b_msg2, W_upd2,
           b_upd2, W_ec1, b_ec1, W_ec2, b_ec2, W_nc1, b_nc1, bn_gamma,
           bn_beta, W_nc2, b_nc2):
    f32 = jnp.float32
    src = edge_index[0]
    dst = edge_index[1]

    feat = jnp.concatenate([
        edge_log_amount[:, None], edge_country_risks[:, None],
        edge_time_since_prevs[:, None], edge_time_gap_between_edges[:, None],
        edge_rolling_tx_count_7d[:, None], edge_rolling_tx_count_30d[:, None],
        edge_ts_encodings], axis=1)
    bank_proj = bank_emb @ W_ee[14:22]
    tx_proj = tx_emb @ W_ee[22:26]
    bank_rows = _sc_gather32(bank_proj, edge_bank_pairs)
    tt2 = edge_tx_types[:, None]

    eaw1, eaw2, eac = pl.pallas_call(
        _edge_dense_body,
        grid=(GE,),
        in_specs=[_eblock(14), _eblock(32), _eblock(1),
                  _full2(16, 32),
                  _full2(14, 32), _full1(32),
                  _full2(32, 128), _full1(128),
                  _full2(32, 64), _full1(64),
                  _full2(32, 64), _full1(64)],
        out_specs=[_eblock(128), _eblock(64), _eblock(64)],
        out_shape=[jax.ShapeDtypeStruct((E, 128), f32),
                   jax.ShapeDtypeStruct((E, 64), f32),
                   jax.ShapeDtypeStruct((E, 64), f32)],
    )(feat, bank_rows, tt2, tx_proj, W_ee[:14], b_ee,
      W_msg1[128:], b_msg1, W_msg2[128:], b_msg2, W_ec1[128:], b_ec1)

    xW1 = pl.pallas_call(
        _mm128_body, grid=(GN,),
        in_specs=[_nblock(128), _full2(128, 128)],
        out_specs=_nblock(128),
        out_shape=jax.ShapeDtypeStruct((N, 128), f32),
    )(x, W_msg1[:128])
    z128 = jnp.zeros((_RST, 128), f32)
    z64 = jnp.zeros((_RST, 64), f32)
    zN = jnp.zeros((2000,), f32)
    ones128 = jnp.ones((_BCH,), f32)
    agg1_p, deg_p = _sc_msg_agg_128(xW1, eaw1, src, dst, z128, zN, ones128)
    agg1 = agg1_p[0, :N] + agg1_p[1, :N]
    deg = deg_p[:N] + deg_p[N:]
    degc = deg[:, None]

    h1, h1W2 = pl.pallas_call(
        _node1_body,
        grid=(GN,),
        in_specs=[_nblock(128), _nblock(128), _nblock(1),
                  _full2(128, 128), _full2(128, 128), _full1(128),
                  _full2(128, 64)],
        out_specs=[_nblock(128), _nblock(64)],
        out_shape=[jax.ShapeDtypeStruct((N, 128), f32),
                   jax.ShapeDtypeStruct((N, 64), f32)],
    )(x, agg1, degc, W_upd1[:128], W_upd1[128:], b_upd1, W_msg2[:128])

    agg2_p = _sc_msg_agg_64(h1W2, eaw2, src, dst, z64, zN, ones128)
    agg2 = agg2_p[0, :N] + agg2_p[1, :N]

    emb, embA, embB = pl.pallas_call(
        _node2_body,
        grid=(GN,),
        in_specs=[_nblock(128), _nblock(64), _nblock(1),
                  _full2(128, 64), _full2(64, 64), _full1(64),
                  _full2(64, 64), _full2(64, 64)],
        out_specs=[_nblock(64), _nblock(64), _nblock(64)],
        out_shape=[jax.ShapeDtypeStruct((N, 64), f32)] * 3,
    )(h1, agg2, degc, W_upd2[:128], W_upd2[128:], b_upd2,
      W_ec1[:64], W_ec1[64:128])

    eh = _sc_edge_eh(embA, embB, eac, src, dst)
    ts2 = edge_unix_ts.astype(f32)[:, None]
    now11 = jnp.max(ts2).reshape(1, 1)
    edge_logits2, probs2, decay2, l302, aged2 = pl.pallas_call(
        _edge_head_body, grid=(GE,),
        in_specs=[_eblock(64), _full2(64, 1), _full2(1, 1),
                  _eblock(1), _full2(1, 1)],
        out_specs=[_eblock(1)] * 5,
        out_shape=[jax.ShapeDtypeStruct((E, 1), f32)] * 5,
    )(eh, W_ec2, b_ec2[:, None], ts2, now11)
    edge_logits = edge_logits2[:, 0]

    z8 = jnp.zeros((_NPAD // _NS, 8), f32)
    adds_p, mm_p = _sc_stats(probs2[:, 0], decay2[:, 0], l302[:, 0],
                             aged2[:, 0], src, dst, z8)
    adds = adds_p[0, :N] + adds_p[1, :N]
    cnt = adds[:, 0]
    sum_prob = adds[:, 1]
    ch_raw = adds[:, 2]
    ws = adds[:, 3]
    wsum = adds[:, 4]
    s30_raw = adds[:, 5]
    sr30 = adds[:, 6]
    c30 = adds[:, 7]
    max_prob = jnp.maximum(mm_p[0, 0, :N], mm_p[1, 0, :N])
    m30 = jnp.maximum(mm_p[0, 1, :N], mm_p[1, 1, :N])
    min_age = jnp.minimum(mm_p[0, 2, :N], mm_p[1, 2, :N])

    stats11 = jnp.stack([cnt, sum_prob, max_prob, ch_raw, ws, wsum, s30_raw,
                         m30, sr30, c30, min_age], axis=1)
    node_logits2 = pl.pallas_call(
        _node_head_body,
        grid=(GN,),
        in_specs=[_nblock(64), _nblock(11),
                  _full2(64, 64), _full2(8, 64), _full1(64), _full1(64),
                  _full1(64), _full2(64, 1), _full2(1, 1)],
        out_specs=_nblock(1),
        out_shape=jax.ShapeDtypeStruct((N, 1), f32),
    )(emb, stats11,
      W_nc1[:64], W_nc1[64:], b_nc1, bn_gamma, bn_beta, W_nc2, b_nc2[:, None])

    return (node_logits2[:, 0], edge_logits)


# speedup vs baseline: 4.5616x; 1.1794x over previous
import functools
import math

import jax
import jax.numpy as jnp
from jax import lax
from jax.experimental import pallas as pl
from jax.experimental.pallas import tpu as pltpu
from jax.experimental.pallas import tpu_sc as plsc

N = 10000
E = 320000
BE = 3200
GE = E // BE


def _edge_dense_body(feat_ref, bank_ref, tt_ref, txp_ref, w14_ref, bee_ref,
                     wm1e_ref, bm1_ref, wm2e_ref, bm2_ref, wecc_ref, bec1_ref,
                     eaw1_ref, eaw2_ref, eac_ref):
    tt = tt_ref[...]
    iota = lax.broadcasted_iota(jnp.int32, (tt.shape[0], 16), 1)
    onehot = (tt == iota).astype(jnp.float32)
    ea = (feat_ref[...] @ w14_ref[...] + bank_ref[...]
          + onehot @ txp_ref[...] + bee_ref[...])
    ea = jnp.maximum(ea, 0.0)
    eaw1_ref[...] = ea @ wm1e_ref[...] + bm1_ref[...]
    eaw2_ref[...] = ea @ wm2e_ref[...] + bm2_ref[...]
    eac_ref[...] = ea @ wecc_ref[...] + bec1_ref[...]


def _mm128_body(a_ref, w_ref, o_ref):
    o_ref[...] = a_ref[...] @ w_ref[...]


def _node1_body(x_ref, agg_ref, deg_ref, wux_ref, wua_ref, bu_ref, wm2h_ref,
                h1_ref, h1w2_ref):
    agg = agg_ref[...] / (deg_ref[...] + 1e-6)
    h1 = jnp.maximum(x_ref[...] @ wux_ref[...] + agg @ wua_ref[...] + bu_ref[...], 0.0)
    h1_ref[...] = h1
    h1w2_ref[...] = h1 @ wm2h_ref[...]


def _node2_body(h1_ref, agg_ref, deg_ref, wux_ref, wua_ref, bu_ref,
                weca_ref, wecb_ref, emb_ref, emba_ref, embb_ref):
    agg = agg_ref[...] / (deg_ref[...] + 1e-6)
    emb = jnp.maximum(h1_ref[...] @ wux_ref[...] + agg @ wua_ref[...] + bu_ref[...], 0.0)
    emb_ref[...] = emb
    emba_ref[...] = emb @ weca_ref[...]
    embb_ref[...] = emb @ wecb_ref[...]


def _edge_head_body(eh_ref, wec2_ref, bec2_ref, ts_ref, now_ref,
                    logit_ref, prob_ref, decay_ref, l30_ref, aged_ref):
    logit = eh_ref[...] @ wec2_ref[...] + bec2_ref[...]
    logit_ref[...] = logit
    prob_ref[...] = 1.0 / (1.0 + jnp.exp(-logit))
    age = jnp.maximum(now_ref[...] - ts_ref[...], 0.0)
    decay_ref[...] = jnp.exp(age * (-1.0 / (30.0 * 86400.0)))
    l30_ref[...] = jnp.where(age <= 30.0 * 86400.0, 1.0, 0.0)
    aged_ref[...] = age * (1.0 / 86400.0)


def _node_head_body(emb_ref, st_ref, wn1e_ref, wn1a_ref, bn1_ref, gam_ref,
                    bet_ref, wn2_ref, bn2_ref, out_ref):
    st = st_ref[...]
    cnt = st[:, 0:1]
    mean_prob = st[:, 1:2] / (cnt + 1e-6)
    maxp = st[:, 2:3]
    max_prob = jnp.where(maxp < -1e8, 0.0, maxp)
    count_high = jnp.log1p(st[:, 3:4])
    decay_weighted = st[:, 4:5] / (st[:, 5:6] + 1e-6)
    s30 = jnp.log1p(st[:, 6:7])
    m30v = st[:, 7:8]
    m30 = jnp.where(m30v < -1e8, 0.0, m30v)
    avg30 = st[:, 8:9] / (st[:, 9:10] + 1e-6)
    tsl = jnp.log1p(jnp.minimum(st[:, 10:11], 90.0)) * (1.0 / math.log1p(90.0))
    stats = (mean_prob, max_prob, count_high, decay_weighted, s30, m30,
             avg30, tsl)
    nh = emb_ref[...] @ wn1e_ref[...] + bn1_ref[...]
    for k, s in enumerate(stats):
        nh = nh + s * wn1a_ref[k:k + 1, :]
    nh = nh * (1.0 / math.sqrt(1.0 + 1e-5)) * gam_ref[...] + bet_ref[...]
    nh = jnp.maximum(nh, 0.0)
    out_ref[...] = nh @ wn2_ref[...] + bn2_ref[...]


BN = 2000
GN = N // BN

_NC, _NS, _NW = 2, 16, 32
_EPT = E // _NW
_BCH = 128
_NFULL = _EPT // _BCH
_TAIL = _EPT - _NFULL * _BCH
_NPAD = 10240
_RPT = _NPAD // _NS
_RST = 128


def _make_sc_msg_agg(D, with_deg):
    f32 = jnp.float32
    mesh = plsc.VectorSubcoreMesh(core_axis_name="c", subcore_axis_name="s",
                                  num_cores=_NC, num_subcores=_NS)
    if with_deg:
        out_type = [jax.ShapeDtypeStruct((_NC, _NPAD, D), f32),
                    jax.ShapeDtypeStruct((_NC * N,), f32)]
    else:
        out_type = jax.ShapeDtypeStruct((_NC, _NPAD, D), f32)
    scratch = [
        pltpu.VMEM((_BCH,), jnp.int32),
        pltpu.VMEM((_BCH,), jnp.int32),
        pltpu.VMEM((_TAIL,), jnp.int32),
        pltpu.VMEM((_TAIL,), jnp.int32),
        pltpu.VMEM((_BCH, D), f32),
        pltpu.VMEM((_BCH, D), f32),
        pltpu.VMEM((_BCH,), f32),
        pltpu.VMEM((2000,), f32),
        pltpu.VMEM_SHARED((_NPAD, D), f32),
        pltpu.VMEM_SHARED((N,), f32),
        pltpu.SemaphoreType.DMA,
        pltpu.SemaphoreType.DMA,
        pltpu.SemaphoreType.DMA,
        pltpu.SemaphoreType.DMA,
    ]

    def body(xw, eaw, srci, dsti, zrows, zdeg, ones, agg_out, *rest):
        if with_deg:
            deg_out = rest[0]
            rest = rest[1:]
        (src_v, dst_v, tsrc_v, tdst_v, g_v, ea_v, ones_v, dstage_v, agg_s,
         deg_s, sem, sem_a, sem_b, sem_c) = rest
        cid = lax.axis_index("c")
        sid = lax.axis_index("s")
        wid = cid * _NS + sid
        base = wid * _EPT

        pltpu.sync_copy(zrows, g_v.at[pl.ds(0, _RST), :])
        for j in range(_RPT // _RST):
            pltpu.sync_copy(g_v.at[pl.ds(0, _RST), :],
                            agg_s.at[pl.ds(sid * _RPT + j * _RST, _RST), :])
        if with_deg:
            pltpu.sync_copy(ones, ones_v)

            @pl.when(sid == 0)
            def _():
                pltpu.sync_copy(zdeg, dstage_v)
                for j in range(N // 2000):
                    pltpu.sync_copy(dstage_v, deg_s.at[pl.ds(j * 2000, 2000)])
        plsc.subcore_barrier()

        def do_chunk(off, sz, sv, dv):
            gd = g_v.at[pl.ds(0, sz), :] if sz != _BCH else g_v
            ed = ea_v.at[pl.ds(0, sz), :] if sz != _BCH else ea_v
            ca = pltpu.async_copy(srci.at[pl.ds(base + off, sz)], sv, sem_a)
            cb = pltpu.async_copy(dsti.at[pl.ds(base + off, sz)], dv, sem_b)
            cc = pltpu.async_copy(eaw.at[pl.ds(base + off, sz), :], ed, sem_c)
            ca.wait()
            cg = pltpu.async_copy(xw.at[sv], gd, sem)
            cb.wait()
            cc.wait()
            cg.wait()

            @pl.loop(0, sz)
            def _(r):
                for c in range(D // 16):
                    s = pl.ds(c * 16, 16)
                    g_v[r, s] = jnp.maximum(g_v[r, s] + ea_v[r, s], 0.0)

            pltpu.sync_copy(gd, agg_s.at[dv], add=True)
            if with_deg:
                ov = ones_v if sz == _BCH else ones_v.at[pl.ds(0, sz)]
                pltpu.sync_copy(ov, deg_s.at[dv], add=True)

        for ch in range(_NFULL):
            do_chunk(ch * _BCH, _BCH, src_v, dst_v)
        do_chunk(_NFULL * _BCH, _TAIL, tsrc_v, tdst_v)

        plsc.subcore_barrier()

        for j in range(_RPT // _RST):
            r0 = sid * _RPT + j * _RST
            pltpu.sync_copy(agg_s.at[pl.ds(r0, _RST), :],
                            g_v.at[pl.ds(0, _RST), :])
            pltpu.sync_copy(g_v.at[pl.ds(0, _RST), :],
                            agg_out.at[cid, pl.ds(r0, _RST), :])
        if with_deg:
            @pl.when(sid == 0)
            def _():
                for j in range(N // 2000):
                    pltpu.sync_copy(deg_s.at[pl.ds(j * 2000, 2000)], dstage_v)
                    pltpu.sync_copy(dstage_v,
                                    deg_out.at[pl.ds(cid * N + j * 2000, 2000)])

    return functools.partial(
        pl.kernel, out_type=out_type, mesh=mesh, scratch_types=scratch,
        compiler_params=pltpu.CompilerParams(use_tc_tiling_on_sc=False),
    )(body)


_sc_msg_agg_128 = _make_sc_msg_agg(128, True)
_sc_msg_agg_64 = _make_sc_msg_agg(64, False)


def _make_sc_edge_eh():
    f32 = jnp.float32
    D = 64
    mesh = plsc.VectorSubcoreMesh(core_axis_name="c", subcore_axis_name="s",
                                  num_cores=_NC, num_subcores=_NS)
    scratch = [
        pltpu.VMEM((_BCH,), jnp.int32),
        pltpu.VMEM((_BCH,), jnp.int32),
        pltpu.VMEM((_TAIL,), jnp.int32),
        pltpu.VMEM((_TAIL,), jnp.int32),
        pltpu.VMEM((_BCH, D), f32),
        pltpu.VMEM((_BCH, D), f32),
        pltpu.VMEM((_BCH, D), f32),
        pltpu.SemaphoreType.DMA,
        pltpu.SemaphoreType.DMA,
        pltpu.SemaphoreType.DMA,
        pltpu.SemaphoreType.DMA,
        pltpu.SemaphoreType.DMA,
    ]

    def body(emba, embb, eac, srci, dsti, eh_out,
             src_v, dst_v, tsrc_v, tdst_v, ga_v, gb_v, ec_v, sema, semb,
             semc, semd, seme):
        cid = lax.axis_index("c")
        sid = lax.axis_index("s")
        wid = cid * _NS + sid
        base = wid * _EPT

        def do_chunk(off, sz, sv, dv):
            gad = ga_v.at[pl.ds(0, sz), :] if sz != _BCH else ga_v
            gbd = gb_v.at[pl.ds(0, sz), :] if sz != _BCH else gb_v
            ecd = ec_v.at[pl.ds(0, sz), :] if sz != _BCH else ec_v
            cs = pltpu.async_copy(srci.at[pl.ds(base + off, sz)], sv, semc)
            cd = pltpu.async_copy(dsti.at[pl.ds(base + off, sz)], dv, semd)
            ce = pltpu.async_copy(eac.at[pl.ds(base + off, sz), :], ecd, seme)
            cs.wait()
            cpa = pltpu.async_copy(emba.at[sv], gad, sema)
            cd.wait()
            cpb = pltpu.async_copy(embb.at[dv], gbd, semb)
            ce.wait()
            cpa.wait()
            cpb.wait()

            @pl.loop(0, sz)
            def _(r):
                for c in range(D // 16):
                    s = pl.ds(c * 16, 16)
                    ga_v[r, s] = jnp.maximum(
                        ga_v[r, s] + gb_v[r, s] + ec_v[r, s], 0.0)

            pltpu.sync_copy(gad, eh_out.at[pl.ds(base + off, sz), :])

        for ch in range(_NFULL):
            do_chunk(ch * _BCH, _BCH, src_v, dst_v)
        do_chunk(_NFULL * _BCH, _TAIL, tsrc_v, tdst_v)

    return functools.partial(
        pl.kernel, out_type=jax.ShapeDtypeStruct((E, 64), f32), mesh=mesh,
        scratch_types=scratch,
        compiler_params=pltpu.CompilerParams(use_tc_tiling_on_sc=False),
    )(body)


_sc_edge_eh = _make_sc_edge_eh()


def _make_sc_gather32():
    f32 = jnp.float32
    D = 32
    mesh = plsc.VectorSubcoreMesh(core_axis_name="c", subcore_axis_name="s",
                                  num_cores=_NC, num_subcores=_NS)
    scratch = [
        pltpu.VMEM((_BCH,), jnp.int32),
        pltpu.VMEM((_TAIL,), jnp.int32),
        pltpu.VMEM((_BCH, D), f32),
        pltpu.SemaphoreType.DMA,
    ]

    def body(table, idxi, rows_out, idx_v, tidx_v, g_v, sem):
        cid = lax.axis_index("c")
        sid = lax.axis_index("s")
        wid = cid * _NS + sid
        base = wid * _EPT

        def do_chunk(off, sz, iv):
            pltpu.sync_copy(idxi.at[pl.ds(base + off, sz)], iv)
            gd = g_v.at[pl.ds(0, sz), :] if sz != _BCH else g_v
            pltpu.async_copy(table.at[iv], gd, sem).wait()
            pltpu.sync_copy(gd, rows_out.at[pl.ds(base + off, sz), :])

        for ch in range(_NFULL):
            do_chunk(ch * _BCH, _BCH, idx_v)
        do_chunk(_NFULL * _BCH, _TAIL, tidx_v)

    return functools.partial(
        pl.kernel, out_type=jax.ShapeDtypeStruct((E, 32), f32), mesh=mesh,
        scratch_types=scratch,
        compiler_params=pltpu.CompilerParams(use_tc_tiling_on_sc=False),
    )(body)


_sc_gather32 = _make_sc_gather32()


def _make_sc_stats():
    f32 = jnp.float32
    i32 = jnp.int32
    RPT = _NPAD // _NS

    mesh = plsc.VectorSubcoreMesh(core_axis_name="c", subcore_axis_name="s",
                                  num_cores=_NC, num_subcores=_NS)
    scratch = [
        pltpu.VMEM((_BCH,), i32),
        pltpu.VMEM((_BCH,), i32),
        pltpu.VMEM((_TAIL,), i32),
        pltpu.VMEM((_TAIL,), i32),
        pltpu.VMEM((_BCH,), f32),
        pltpu.VMEM((_BCH,), f32),
        pltpu.VMEM((_BCH,), f32),
        pltpu.VMEM((_BCH,), f32),
        pltpu.VMEM((_BCH, 8), f32),
        pltpu.VMEM((_NPAD,), f32),
        pltpu.VMEM((_NPAD,), f32),
        pltpu.VMEM((_NPAD,), f32),
        pltpu.VMEM((_NS, RPT), f32),
        pltpu.VMEM((RPT,), f32),
        pltpu.VMEM((RPT, 8), f32),
        pltpu.VMEM((48,), i32),
        pltpu.VMEM((48,), f32),
        pltpu.VMEM_SHARED((_NPAD, 8), f32),
        pltpu.VMEM_SHARED((3, _NS, _NPAD), f32),
        [pltpu.SemaphoreType.DMA] * 6,
    ]

    def body(probs, decay, last30, aged, srci, dsti, z8,
             adds_out, mm_out,
             src_v, dst_v, tsrc_v, tdst_v, p_v, dc_v, l3_v, ag_v, rows_v,
             mx_v, m3_v, mn_v, red_v, out_v, zst_v, kw_v, vw_v,
             adds_s, mm_s, sems):
        cid = lax.axis_index("c")
        sid = lax.axis_index("s")
        wid = cid * _NS + sid
        base = wid * _EPT
        iota = lax.iota(i32, 16)

        pltpu.sync_copy(z8, zst_v)
        pltpu.sync_copy(zst_v, adds_s.at[pl.ds(sid * RPT, RPT), :])

        @pl.loop(0, _NPAD // 16)
        def _(i):
            s = pl.ds(i * 16, 16)
            mx_v[s] = jnp.full((16,), -1e9, f32)
            m3_v[s] = jnp.full((16,), -1e9, f32)
            mn_v[s] = jnp.full((16,), 9999.0, f32)

        kw_v[pl.ds(0, 16)] = jnp.full((16,), -1, i32)
        kw_v[pl.ds(32, 16)] = jnp.full((16,), -2, i32)
        plsc.subcore_barrier()

        def segreduce(arr, kv, vv, is_min):
            ks, vs = plsc.sort_key_val(kv, vv)
            op = jnp.minimum if is_min else jnp.maximum
            kw_v[pl.ds(16, 16)] = ks
            for sh in (1, 2, 4, 8):
                vw_v[pl.ds(16, 16)] = vs
                same = kw_v[pl.ds(16 - sh, 16)] == ks
                shifted = vw_v[pl.ds(16 - sh, 16)]
                vs = jnp.where(same, op(vs, shifted), vs)
            knext = kw_v[pl.ds(17, 16)]
            last = knext != ks
            cur = plsc.load_gather(arr, [ks], mask=last)
            plsc.store_scatter(arr, [ks], op(cur, vs), mask=last)

        def do_chunk(off, sz, sv, dv):
            srcs = (srci, dsti, probs, decay, last30, aged)
            dsts = (sv, dv, p_v.at[pl.ds(0, sz)], dc_v.at[pl.ds(0, sz)],
                    l3_v.at[pl.ds(0, sz)], ag_v.at[pl.ds(0, sz)])
            cps = [pltpu.async_copy(s.at[pl.ds(base + off, sz)], t, sems[i])
                   for i, (s, t) in enumerate(zip(srcs, dsts))]
            for cp in cps:
                cp.wait()
            @pl.loop(0, sz // 16)
            def _(g):
                sl = pl.ds(g * 16, 16)
                eids = iota + g * 16
                p16 = p_v[sl]
                dc16 = dc_v[sl]
                l316 = l3_v[sl]
                ag16 = ag_v[sl]
                high = jnp.where(p16 >= 0.7, 1.0, 0.0).astype(f32)
                vals = (jnp.full((16,), 1.0, f32), p16, high, p16 * dc16,
                        dc16, high * l316, p16 * l316, l316)
                for k, v in enumerate(vals):
                    plsc.store_scatter(rows_v, [eids, jnp.full((16,), k, i32)], v)
                vm30 = jnp.where(l316 > 0.5, p16, jnp.full((16,), -1e9, f32))
                vmin = jnp.where(p16 >= 0.7, ag16, jnp.full((16,), 1e30, f32))
                for kv in (sv[sl], dv[sl]):
                    segreduce(mx_v, kv, p16, False)
                    segreduce(m3_v, kv, vm30, False)
                    segreduce(mn_v, kv, vmin, True)
            rd = rows_v if sz == _BCH else rows_v.at[pl.ds(0, sz), :]
            pltpu.sync_copy(rd, adds_s.at[sv], add=True)
            pltpu.sync_copy(rd, adds_s.at[dv], add=True)

        @pl.loop(0, _NFULL)
        def _(ch):
            do_chunk(pl.multiple_of(ch * _BCH, _BCH), _BCH, src_v, dst_v)

        do_chunk(_NFULL * _BCH, _TAIL, tsrc_v, tdst_v)

        pltpu.sync_copy(mx_v, mm_s.at[0, sid, :])
        pltpu.sync_copy(m3_v, mm_s.at[1, sid, :])
        pltpu.sync_copy(mn_v, mm_s.at[2, sid, :])
        plsc.subcore_barrier()

        pltpu.sync_copy(adds_s.at[pl.ds(sid * RPT, RPT), :], zst_v)
        pltpu.sync_copy(zst_v, adds_out.at[cid, pl.ds(sid * RPT, RPT), :])
        for st in range(3):
            pltpu.sync_copy(mm_s.at[st, :, pl.ds(sid * RPT, RPT)], red_v)
            op = jnp.minimum if st == 2 else jnp.maximum

            @pl.loop(0, RPT // 16)
            def _(j):
                sl = pl.ds(j * 16, 16)
                acc = red_v[0, sl]
                for r in range(1, _NS):
                    acc = op(acc, red_v[r, sl])
                out_v[sl] = acc

            pltpu.sync_copy(out_v, mm_out.at[cid, st, pl.ds(sid * RPT, RPT)])

    return functools.partial(
        pl.kernel,
        out_type=[jax.ShapeDtypeStruct((_NC, _NPAD, 8), f32),
                  jax.ShapeDtypeStruct((_NC, 3, _NPAD), f32)],
        mesh=mesh, scratch_types=scratch,
        compiler_params=pltpu.CompilerParams(use_tc_tiling_on_sc=False,
                                             needs_layout_passes=False),
    )(body)


_sc_stats = _make_sc_stats()


def _eblock(d):
    return pl.BlockSpec((BE, d), lambda i: (i, 0))


def _nblock(d):
    return pl.BlockSpec((BN, d), lambda i: (i, 0))


def _full2(a, b):
    return pl.BlockSpec((a, b), lambda i: (0, 0))


def _full1(a):
    return pl.BlockSpec((a,), lambda i: (0,))


def kernel(x, edge_index, edge_log_amount, edge_ts_encodings, edge_bank_pairs,
           edge_tx_types, edge_country_risks, edge_time_since_prevs,
           edge_time_gap_between_edges, edge_rolling_tx_count_7d,
           edge_rolling_tx_count_30d, edge_unix_ts, bank_emb, tx_emb, W_ee,
           b_ee, W_msg1, b_msg1, W_upd1, b_upd1, W_msg2, b_msg2, W_upd2,
           b_upd2, W_ec1, b_ec1, W_ec2, b_ec2, W_nc1, b_nc1, bn_gamma,
           bn_beta, W_nc2, b_nc2):
    f32 = jnp.float32
    src = edge_index[0]
    dst = edge_index[1]

    feat = jnp.concatenate([
        edge_log_amount[:, None], edge_country_risks[:, None],
        edge_time_since_prevs[:, None], edge_time_gap_between_edges[:, None],
        edge_rolling_tx_count_7d[:, None], edge_rolling_tx_count_30d[:, None],
        edge_ts_encodings], axis=1)
    bank_proj = bank_emb @ W_ee[14:22]
    tx_proj = tx_emb @ W_ee[22:26]
    bank_rows = _sc_gather32(bank_proj, edge_bank_pairs)
    tt2 = edge_tx_types[:, None]

    eaw1, eaw2, eac = pl.pallas_call(
        _edge_dense_body,
        grid=(GE,),
        in_specs=[_eblock(14), _eblock(32), _eblock(1),
                  _full2(16, 32),
                  _full2(14, 32), _full1(32),
                  _full2(32, 128), _full1(128),
                  _full2(32, 64), _full1(64),
                  _full2(32, 64), _full1(64)],
        out_specs=[_eblock(128), _eblock(64), _eblock(64)],
        out_shape=[jax.ShapeDtypeStruct((E, 128), f32),
                   jax.ShapeDtypeStruct((E, 64), f32),
                   jax.ShapeDtypeStruct((E, 64), f32)],
    )(feat, bank_rows, tt2, tx_proj, W_ee[:14], b_ee,
      W_msg1[128:], b_msg1, W_msg2[128:], b_msg2, W_ec1[128:], b_ec1)

    xW1 = pl.pallas_call(
        _mm128_body, grid=(GN,),
        in_specs=[_nblock(128), _full2(128, 128)],
        out_specs=_nblock(128),
        out_shape=jax.ShapeDtypeStruct((N, 128), f32),
    )(x, W_msg1[:128])
    z128 = jnp.zeros((_RST, 128), f32)
    z64 = jnp.zeros((_RST, 64), f32)
    zN = jnp.zeros((2000,), f32)
    ones128 = jnp.ones((_BCH,), f32)
    agg1_p, deg_p = _sc_msg_agg_128(xW1, eaw1, src, dst, z128, zN, ones128)
    agg1 = agg1_p[0, :N] + agg1_p[1, :N]
    deg = deg_p[:N] + deg_p[N:]
    degc = deg[:, None]

    h1, h1W2 = pl.pallas_call(
        _node1_body,
        grid=(GN,),
        in_specs=[_nblock(128), _nblock(128), _nblock(1),
                  _full2(128, 128), _full2(128, 128), _full1(128),
                  _full2(128, 64)],
        out_specs=[_nblock(128), _nblock(64)],
        out_shape=[jax.ShapeDtypeStruct((N, 128), f32),
                   jax.ShapeDtypeStruct((N, 64), f32)],
    )(x, agg1, degc, W_upd1[:128], W_upd1[128:], b_upd1, W_msg2[:128])

    agg2_p = _sc_msg_agg_64(h1W2, eaw2, src, dst, z64, zN, ones128)
    agg2 = agg2_p[0, :N] + agg2_p[1, :N]

    emb, embA, embB = pl.pallas_call(
        _node2_body,
        grid=(GN,),
        in_specs=[_nblock(128), _nblock(64), _nblock(1),
                  _full2(128, 64), _full2(64, 64), _full1(64),
                  _full2(64, 64), _full2(64, 64)],
        out_specs=[_nblock(64), _nblock(64), _nblock(64)],
        out_shape=[jax.ShapeDtypeStruct((N, 64), f32)] * 3,
    )(h1, agg2, degc, W_upd2[:128], W_upd2[128:], b_upd2,
      W_ec1[:64], W_ec1[64:128])

    eh = _sc_edge_eh(embA, embB, eac, src, dst)
    ts2 = edge_unix_ts.astype(f32)[:, None]
    now11 = jnp.max(ts2).reshape(1, 1)
    edge_logits2, probs2, decay2, l302, aged2 = pl.pallas_call(
        _edge_head_body, grid=(GE,),
        in_specs=[_eblock(64), _full2(64, 1), _full2(1, 1),
                  _eblock(1), _full2(1, 1)],
        out_specs=[_eblock(1)] * 5,
        out_shape=[jax.ShapeDtypeStruct((E, 1), f32)] * 5,
    )(eh, W_ec2, b_ec2[:, None], ts2, now11)
    edge_logits = edge_logits2[:, 0]

    z8 = jnp.zeros((_NPAD // _NS, 8), f32)
    adds_p, mm_p = _sc_stats(probs2[:, 0], decay2[:, 0], l302[:, 0],
                             aged2[:, 0], src, dst, z8)
    adds = adds_p[0, :N] + adds_p[1, :N]
    cnt = adds[:, 0]
    sum_prob = adds[:, 1]
    ch_raw = adds[:, 2]
    ws = adds[:, 3]
    wsum = adds[:, 4]
    s30_raw = adds[:, 5]
    sr30 = adds[:, 6]
    c30 = adds[:, 7]
    max_prob = jnp.maximum(mm_p[0, 0, :N], mm_p[1, 0, :N])
    m30 = jnp.maximum(mm_p[0, 1, :N], mm_p[1, 1, :N])
    min_age = jnp.minimum(mm_p[0, 2, :N], mm_p[1, 2, :N])

    stats11 = jnp.stack([cnt, sum_prob, max_prob, ch_raw, ws, wsum, s30_raw,
                         m30, sr30, c30, min_age], axis=1)
    node_logits2 = pl.pallas_call(
        _node_head_body,
        grid=(GN,),
        in_specs=[_nblock(64), _nblock(11),
                  _full2(64, 64), _full2(8, 64), _full1(64), _full1(64),
                  _full1(64), _full2(64, 1), _full2(1, 1)],
        out_specs=_nblock(1),
        out_shape=jax.ShapeDtypeStruct((N, 1), f32),
    )(emb, stats11,
      W_nc1[:64], W_nc1[64:], b_nc1, bn_gamma, bn_beta, W_nc2, b_nc2[:, None])

    return (node_logits2[:, 0], edge_logits)
